# baseline (speedup 1.0000x reference)
import functools
import jax
import jax.numpy as jnp
from jax import lax
from jax.experimental import pallas as pl
from jax.experimental.pallas import tpu as pltpu

N = 10000
E = 320000
HI = jax.lax.Precision.HIGHEST


def _mm_body(nx, act, *refs):
    xs = refs[:nx]
    ws = refs[nx:2 * nx]
    b = refs[2 * nx]
    o = refs[2 * nx + 1]
    acc = jnp.zeros(o.shape, jnp.float32)
    for x, w in zip(xs, ws):
        acc = acc + jax.lax.dot_general(
            x[...], w[...], (((1,), (0,)), ((), ())),
            precision=HI, preferred_element_type=jnp.float32)
    acc = acc + b[...]
    if act == 'relu':
        acc = jnp.maximum(acc, 0.0)
    elif act == 'leaky':
        acc = jnp.where(acc >= 0, acc, 0.01 * acc)
    o[...] = acc


def _mm(xs_ws, b, act=None, bm=2000):
    xs = [x for x, _ in xs_ws]
    ws = [w for _, w in xs_ws]
    m = xs[0].shape[0]
    nout = ws[0].shape[1]
    grid = (m // bm,)
    in_specs = (
        [pl.BlockSpec((bm, x.shape[1]), lambda i: (i, 0)) for x in xs]
        + [pl.BlockSpec(w.shape, lambda i: (0, 0)) for w in ws]
        + [pl.BlockSpec((1, nout), lambda i: (0, 0))]
    )
    return pl.pallas_call(
        functools.partial(_mm_body, len(xs), act),
        grid=grid,
        in_specs=in_specs,
        out_specs=pl.BlockSpec((bm, nout), lambda i: (i, 0)),
        out_shape=jax.ShapeDtypeStruct((m, nout), jnp.float32),
    )(*xs, *ws, b.reshape(1, -1))


def _seg_sum(vals, idx, n):
    return jax.ops.segment_sum(vals, idx, num_segments=n)


def _seg_max(vals, idx, n):
    return jax.ops.segment_max(vals, idx, num_segments=n)


def _spmm(x, s_idx, r_idx, w=None):
    rows = x[s_idx]
    if w is not None:
        rows = rows * w[:, None]
    return _seg_sum(rows, r_idx, x.shape[0])


def _gat_weights(a_s, a_r, base, s_idx, r_idx):
    logits = a_s[s_idx] + a_r[r_idx] + base
    mx = _seg_max(logits, r_idx, N)
    e = jnp.exp(logits - mx[r_idx])
    den = _seg_sum(e, r_idx, N)
    return e / den[r_idx]


def kernel(nodes, edges, senders, receivers, globals_,
           W_gcn1, b_gcn1, W_gcn2, b_gcn2,
           Wq1, bq1, Wl1, bl1,
           Wq2, bq2, Wl2, bl2, Wn2, bn2,
           We, be, Wn, bn, Wg, bg):
    s_idx = senders.astype(jnp.int32)
    r_idx = receivers.astype(jnp.int32)
    ones = jnp.ones((E,), jnp.float32)

    send_deg = _seg_sum(ones, s_idx, N) + 1.0
    recv_deg = _seg_sum(ones, r_idx, N) + 1.0
    ssc = jax.lax.rsqrt(jnp.maximum(send_deg, 1.0))[:, None]
    rsc = jax.lax.rsqrt(jnp.maximum(recv_deg, 1.0))[:, None]

    h = _mm([(nodes, W_gcn1)], b_gcn1, act='relu') * ssc
    h = (h + _spmm(h, s_idx, r_idx)) * rsc
    h = _mm([(h, W_gcn2)], b_gcn2, act='relu') * ssc
    h = (h + _spmm(h, s_idx, r_idx)) * rsc

    q = _mm([(h, Wq1)], bq1)
    wl_s, wl_r, wl_e = Wl1[:256, 0], Wl1[256:512, 0], Wl1[512:, 0]
    a_s = q @ wl_s
    a_r = q @ wl_r
    base = edges @ wl_e + bl1[0]
    w1 = _gat_weights(a_s, a_r, base, s_idx, r_idx)
    h = _spmm(q, s_idx, r_idx, w1)
    h = jnp.where(h >= 0, h, 0.01 * h)

    q = _mm([(h, Wq2)], bq2)
    wl_s, wl_r, wl_e = Wl2[:256, 0], Wl2[256:512, 0], Wl2[512:, 0]
    a_s = q @ wl_s
    a_r = q @ wl_r
    base = edges @ wl_e + bl2[0]
    w2 = _gat_weights(a_s, a_r, base, s_idx, r_idx)
    agg = _spmm(q, s_idx, r_idx, w2)
    h = _mm([(agg, Wn2)], bn2)

    We_e, We_s, We_r = We[:16], We[16:272], We[272:528]
    ps = _mm([(h, We_s)], jnp.zeros((256,), jnp.float32))
    pr = _mm([(h, We_r)], jnp.zeros((256,), jnp.float32))
    eq = _mm([(edges, We_e)], be, bm=4000)
    new_edges = jnp.maximum(eq + ps[s_idx] + pr[r_idx], 0.0)
    sent_agg = _seg_sum(new_edges, s_idx, N)
    recv_agg = _seg_sum(new_edges, r_idx, N)

    Wn_h, Wn_s, Wn_r = Wn[:256], Wn[256:512], Wn[512:768]
    new_nodes = _mm([(h, Wn_h), (sent_agg, Wn_s), (recv_agg, Wn_r)], bn,
                    act='relu')

    node_attr = jnp.sum(new_nodes, axis=0, keepdims=True)
    edge_attr = jnp.sum(new_edges, axis=0, keepdims=True)
    g = jnp.zeros((1, 64), jnp.float32)
    feats = jnp.concatenate([node_attr, edge_attr, g], axis=1)
    logits = jnp.maximum(feats @ Wg + bg, 0.0)
    return logits


# speedup vs baseline: 3.3983x; 3.3983x over previous
import functools
import jax
import jax.numpy as jnp
from jax import lax
from jax.experimental import pallas as pl
from jax.experimental.pallas import tpu as pltpu
from jax.experimental.pallas import tpu_sc as plsc

N = 10000
E = 320000
NP = 10240
NT = 16
C = 80
EPT = E // NT
RPT = NP // NT
HI = jax.lax.Precision.HIGHEST
_MESH = dict(core_axis_name="c", subcore_axis_name="s",
             num_cores=2, num_subcores=NT)


def _zero_vmem(ref, n):
    z = jnp.zeros((16,), jnp.float32)

    def zr(i, _):
        ref[pl.ds(i * 16, 16)] = z
        return 0
    lax.fori_loop(0, n // 16, zr, 0)


def _sc_spmm_call(weighted):
    mesh = plsc.VectorSubcoreMesh(**_MESH)
    scratch = [
        pltpu.VMEM((C,), jnp.int32),
        pltpu.VMEM((C,), jnp.int32),
        pltpu.VMEM((C, 128), jnp.float32),
        pltpu.VMEM((C,), jnp.float32),
        pltpu.VMEM_SHARED((NP, 128), jnp.float32),
        pltpu.SemaphoreType.DMA,
    ]

    def body(x2_h, s_h, r_h, w_h, ya_h, yb_h, sv, rv, rows, wv, acc, sem):
        c = lax.axis_index("c")
        t = lax.axis_index("s")

        def zrow(e, _):
            for k in range(8):
                rows[e, pl.ds(k * 16, 16)] = jnp.zeros((16,), jnp.float32)
            return 0
        lax.fori_loop(0, C, zrow, 0)
        for j in range(RPT // C):
            pltpu.sync_copy(rows, acc.at[pl.ds(t * RPT + j * C, C)])
        plsc.subcore_barrier()

        def chunk(k, _):
            base = t * EPT + k * C
            pltpu.sync_copy(s_h.at[pl.ds(base, C)], sv)
            pltpu.sync_copy(r_h.at[pl.ds(base, C)], rv)

            def adj(i, _):
                sl = pl.ds(i * 16, 16)
                sv[sl] = sv[sl] * 2 + c
                return 0
            lax.fori_loop(0, C // 16, adj, 0)
            pltpu.async_copy(x2_h.at[sv], rows, sem).wait()
            if weighted:
                pltpu.sync_copy(w_h.at[pl.ds(base, C)], wv)

                def scale(i, _):
                    wvec = wv[pl.ds(i * 16, 16)]
                    for j in range(16):
                        we = wvec[j]
                        for k2 in range(8):
                            sl = pl.ds(k2 * 16, 16)
                            rows[i * 16 + j, sl] = rows[i * 16 + j, sl] * we
                    return 0
                lax.fori_loop(0, C // 16, scale, 0)
            pltpu.sync_copy(rows, acc.at[rv], add=True)
            return 0
        lax.fori_loop(0, EPT // C, chunk, 0)
        plsc.subcore_barrier()
        sl = pl.ds(t * RPT, RPT)

        @pl.when(c == 0)
        def _():
            pltpu.sync_copy(acc.at[sl], ya_h.at[sl])

        @pl.when(c == 1)
        def _():
            pltpu.sync_copy(acc.at[sl], yb_h.at[sl])

    return pl.kernel(
        body,
        out_type=[jax.ShapeDtypeStruct((NP, 128), jnp.float32)] * 2,
        mesh=mesh, scratch_types=scratch,
        compiler_params=pltpu.CompilerParams(needs_layout_passes=False))


def _sc_spmm(x, s_idx, r_idx, w=None):
    x2 = x.reshape(2 * N, 128)
    if w is None:
        w = jnp.zeros((8,), jnp.float32)
    ya, yb = _sc_spmm_call(w.shape[0] == E)(x2, s_idx, r_idx, w)
    return jnp.concatenate([ya[:N], yb[:N]], axis=1)


def _sc_gatw_call():
    mesh = plsc.VectorSubcoreMesh(**_MESH)
    HALF = EPT // 2
    scratch = [
        pltpu.VMEM((NP,), jnp.float32),
        pltpu.VMEM((NP,), jnp.float32),
        pltpu.VMEM((NP,), jnp.float32),
        pltpu.VMEM((EPT,), jnp.float32),
        pltpu.VMEM((C,), jnp.float32),
        pltpu.VMEM((C,), jnp.int32),
        pltpu.VMEM((C,), jnp.int32),
        pltpu.VMEM((C,), jnp.float32),
        pltpu.VMEM((RPT,), jnp.float32),
        pltpu.VMEM((RPT,), jnp.float32),
        pltpu.VMEM_SHARED((NT, NP), jnp.float32),
        pltpu.VMEM_SHARED((NP,), jnp.float32),
    ]

    def body(as_h, ar_h, b_h, s_h, r_h, w_h,
             asv, arv, den, exv, bb, sv, rv, wbuf, tmpv, comb,
             spm, spm_den):
        c = lax.axis_index("c")
        t = lax.axis_index("s")
        pltpu.sync_copy(as_h, asv)
        pltpu.sync_copy(ar_h, arv)
        _zero_vmem(den, NP)

        def chunk(k, _):
            base = t * EPT + k * C
            pltpu.sync_copy(s_h.at[pl.ds(base, C)], sv)
            pltpu.sync_copy(r_h.at[pl.ds(base, C)], rv)
            pltpu.sync_copy(b_h.at[pl.ds(base, C)], bb)

            def grp(i, _):
                sl = pl.ds(i * 16, 16)
                svec = sv[sl]
                rvec = rv[sl]
                sa = plsc.load_gather(asv, [svec])
                sr = plsc.load_gather(arv, [rvec])
                ex16 = jnp.exp(sa + sr + bb[sl])
                exv[pl.ds(k * C + i * 16, 16)] = ex16
                plsc.addupdate_scatter(den, [rvec], ex16)
                return 0
            lax.fori_loop(0, C // 16, grp, 0)
            return 0
        lax.fori_loop(0, EPT // C, chunk, 0)

        pltpu.sync_copy(den, spm.at[t])
        plsc.subcore_barrier()
        _zero_vmem(comb, RPT)
        for p in range(NT):
            pltpu.sync_copy(spm.at[p, pl.ds(t * RPT, RPT)], tmpv)

            def acc(i, _):
                sl = pl.ds(i * 16, 16)
                comb[sl] = comb[sl] + tmpv[sl]
                return 0
            lax.fori_loop(0, RPT // 16, acc, 0)
        pltpu.sync_copy(comb, spm_den.at[pl.ds(t * RPT, RPT)])
        plsc.subcore_barrier()
        pltpu.sync_copy(spm_den, den)

        def chunk2(k, _):
            off = c * HALF + k * C
            base = t * EPT + off
            pltpu.sync_copy(r_h.at[pl.ds(base, C)], rv)

            def grp2(i, _):
                sl = pl.ds(i * 16, 16)
                rvec = rv[sl]
                dv = plsc.load_gather(den, [rvec])
                ex16 = exv[pl.ds(off + i * 16, 16)]
                wbuf[sl] = ex16 / dv
                return 0
            lax.fori_loop(0, C // 16, grp2, 0)
            pltpu.sync_copy(wbuf, w_h.at[pl.ds(base, C)])
            return 0
        lax.fori_loop(0, HALF // C, chunk2, 0)

    return pl.kernel(
        body,
        out_type=jax.ShapeDtypeStruct((E,), jnp.float32),
        mesh=mesh, scratch_types=scratch,
        compiler_params=pltpu.CompilerParams(needs_layout_passes=False))


def _sc_gatw(a_s, a_r, base, s_idx, r_idx):
    asp = jnp.pad(a_s, (0, NP - N))
    arp = jnp.pad(a_r, (0, NP - N))
    return _sc_gatw_call()(asp, arp, base, s_idx, r_idx)


def _edge_base(edges, wl_e):
    wpad = jnp.zeros((16, 128), jnp.float32).at[:, 0].set(wl_e)
    out = _mm([(edges, wpad)], jnp.zeros((128,), jnp.float32), bm=4000)
    return out[:, 0]


def _sc_degrees_call():
    mesh = plsc.VectorSubcoreMesh(**_MESH)
    scratch = [
        pltpu.VMEM((NP,), jnp.float32),
        pltpu.VMEM((C,), jnp.int32),
        pltpu.VMEM((RPT,), jnp.float32),
        pltpu.VMEM((RPT,), jnp.float32),
        pltpu.VMEM_SHARED((NT, NP), jnp.float32),
    ]

    def body(s_h, r_h, sd_h, rd_h, part, iv, tmpv, comb, spm):
        c = lax.axis_index("c")
        t = lax.axis_index("s")

        def hist(idx_h, out_h):
            _zero_vmem(part, NP)
            ones16 = jnp.ones((16,), jnp.float32)

            def chunk(k, _):
                pltpu.sync_copy(idx_h.at[pl.ds(t * EPT + k * C, C)], iv)

                def grp(i, _):
                    plsc.addupdate_scatter(part, [iv[pl.ds(i * 16, 16)]],
                                           ones16)
                    return 0
                lax.fori_loop(0, C // 16, grp, 0)
                return 0
            lax.fori_loop(0, EPT // C, chunk, 0)
            pltpu.sync_copy(part, spm.at[t])
            plsc.subcore_barrier()
            _zero_vmem(comb, RPT)
            for p in range(NT):
                pltpu.sync_copy(spm.at[p, pl.ds(t * RPT, RPT)], tmpv)

                def acc(i, _):
                    sl = pl.ds(i * 16, 16)
                    comb[sl] = comb[sl] + tmpv[sl]
                    return 0
                lax.fori_loop(0, RPT // 16, acc, 0)
            pltpu.sync_copy(comb, out_h.at[pl.ds(t * RPT, RPT)])

        @pl.when(c == 0)
        def _():
            hist(s_h, sd_h)

        @pl.when(c == 1)
        def _():
            hist(r_h, rd_h)

    return pl.kernel(
        body,
        out_type=[jax.ShapeDtypeStruct((NP,), jnp.float32)] * 2,
        mesh=mesh, scratch_types=scratch,
        compiler_params=pltpu.CompilerParams(needs_layout_passes=False))


def _mm_body(nx, act, *refs):
    xs = refs[:nx]
    ws = refs[nx:2 * nx]
    b = refs[2 * nx]
    o = refs[2 * nx + 1]
    acc = jnp.zeros(o.shape, jnp.float32)
    for x, w in zip(xs, ws):
        acc = acc + jax.lax.dot_general(
            x[...], w[...], (((1,), (0,)), ((), ())),
            precision=HI, preferred_element_type=jnp.float32)
    acc = acc + b[...]
    if act == 'relu':
        acc = jnp.maximum(acc, 0.0)
    elif act == 'leaky':
        acc = jnp.where(acc >= 0, acc, 0.01 * acc)
    o[...] = acc


def _mm(xs_ws, b, act=None, bm=2000):
    xs = [x for x, _ in xs_ws]
    ws = [w for _, w in xs_ws]
    m = xs[0].shape[0]
    nout = ws[0].shape[1]
    grid = (m // bm,)
    in_specs = (
        [pl.BlockSpec((bm, x.shape[1]), lambda i: (i, 0)) for x in xs]
        + [pl.BlockSpec(w.shape, lambda i: (0, 0)) for w in ws]
        + [pl.BlockSpec((1, nout), lambda i: (0, 0))]
    )
    return pl.pallas_call(
        functools.partial(_mm_body, len(xs), act),
        grid=grid,
        in_specs=in_specs,
        out_specs=pl.BlockSpec((bm, nout), lambda i: (i, 0)),
        out_shape=jax.ShapeDtypeStruct((m, nout), jnp.float32),
    )(*xs, *ws, b.reshape(1, -1))


def _seg_sum(vals, idx, n):
    return jax.ops.segment_sum(vals, idx, num_segments=n)


def _seg_max(vals, idx, n):
    return jax.ops.segment_max(vals, idx, num_segments=n)


def _spmm(x, s_idx, r_idx, w=None):
    rows = x[s_idx]
    if w is not None:
        rows = rows * w[:, None]
    return _seg_sum(rows, r_idx, x.shape[0])


def _gat_weights(a_s, a_r, base, s_idx, r_idx):
    logits = a_s[s_idx] + a_r[r_idx] + base
    mx = _seg_max(logits, r_idx, N)
    e = jnp.exp(logits - mx[r_idx])
    den = _seg_sum(e, r_idx, N)
    return e / den[r_idx]


def kernel(nodes, edges, senders, receivers, globals_,
           W_gcn1, b_gcn1, W_gcn2, b_gcn2,
           Wq1, bq1, Wl1, bl1,
           Wq2, bq2, Wl2, bl2, Wn2, bn2,
           We, be, Wn, bn, Wg, bg):
    s_idx = senders.astype(jnp.int32)
    r_idx = receivers.astype(jnp.int32)
    sdeg, rdeg = _sc_degrees_call()(s_idx, r_idx)
    ssc = jax.lax.rsqrt(sdeg[:N, None] + 1.0)
    rsc = jax.lax.rsqrt(rdeg[:N, None] + 1.0)

    h = _mm([(nodes, W_gcn1)], b_gcn1, act='relu') * ssc
    h = (h + _sc_spmm(h, s_idx, r_idx)) * rsc
    h = _mm([(h, W_gcn2)], b_gcn2, act='relu') * ssc
    h = (h + _sc_spmm(h, s_idx, r_idx)) * rsc

    q = _mm([(h, Wq1)], bq1)
    wl_s, wl_r, wl_e = Wl1[:256, 0], Wl1[256:512, 0], Wl1[512:, 0]
    a_s = q @ wl_s + bl1[0]
    a_r = q @ wl_r
    w1 = _sc_gatw(a_s, a_r, _edge_base(edges, wl_e), s_idx, r_idx)
    h = _sc_spmm(q, s_idx, r_idx, w1)
    h = jnp.where(h >= 0, h, 0.01 * h)

    q = _mm([(h, Wq2)], bq2)
    wl_s, wl_r, wl_e = Wl2[:256, 0], Wl2[256:512, 0], Wl2[512:, 0]
    a_s = q @ wl_s + bl2[0]
    a_r = q @ wl_r
    w2 = _sc_gatw(a_s, a_r, _edge_base(edges, wl_e), s_idx, r_idx)
    agg = _sc_spmm(q, s_idx, r_idx, w2)
    h = _mm([(agg, Wn2)], bn2)

    We_e, We_s, We_r = We[:16], We[16:272], We[272:528]
    ps = _mm([(h, We_s)], jnp.zeros((256,), jnp.float32))
    pr = _mm([(h, We_r)], jnp.zeros((256,), jnp.float32))
    eq = _mm([(edges, We_e)], be, bm=4000)
    new_edges = jnp.maximum(eq + ps[s_idx] + pr[r_idx], 0.0)
    sent_agg = _seg_sum(new_edges, s_idx, N)
    recv_agg = _seg_sum(new_edges, r_idx, N)

    Wn_h, Wn_s, Wn_r = Wn[:256], Wn[256:512], Wn[512:768]
    new_nodes = _mm([(h, Wn_h), (sent_agg, Wn_s), (recv_agg, Wn_r)], bn,
                    act='relu')

    node_attr = jnp.sum(new_nodes, axis=0, keepdims=True)
    edge_attr = jnp.sum(new_edges, axis=0, keepdims=True)
    g = jnp.zeros((1, 64), jnp.float32)
    feats = jnp.concatenate([node_attr, edge_attr, g], axis=1)
    logits = jnp.maximum(feats @ Wg + bg, 0.0)
    return logits


# speedup vs baseline: 4.4356x; 1.3053x over previous
import functools
import jax
import jax.numpy as jnp
from jax import lax
from jax.experimental import pallas as pl
from jax.experimental.pallas import tpu as pltpu
from jax.experimental.pallas import tpu_sc as plsc

N = 10000
E = 320000
NP = 10240
NT = 16
C = 80
EPT = E // NT
RPT = NP // NT
HI = jax.lax.Precision.HIGHEST
_MESH = dict(core_axis_name="c", subcore_axis_name="s",
             num_cores=2, num_subcores=NT)


def _zero_vmem(ref, n):
    z = jnp.zeros((16,), jnp.float32)

    def zr(i, _):
        ref[pl.ds(i * 16, 16)] = z
        return 0
    lax.fori_loop(0, n // 16, zr, 0)


def _sc_spmm_call(weighted):
    mesh = plsc.VectorSubcoreMesh(**_MESH)
    scratch = [
        pltpu.VMEM((C,), jnp.int32),
        pltpu.VMEM((C,), jnp.int32),
        pltpu.VMEM((C, 128), jnp.float32),
        pltpu.VMEM((C,), jnp.float32),
        pltpu.VMEM_SHARED((NP, 128), jnp.float32),
        pltpu.SemaphoreType.DMA,
    ]

    def body(x2_h, s_h, r_h, w_h, ya_h, yb_h, sv, rv, rows, wv, acc, sem):
        c = lax.axis_index("c")
        t = lax.axis_index("s")

        def zrow(e, _):
            for k in range(8):
                rows[e, pl.ds(k * 16, 16)] = jnp.zeros((16,), jnp.float32)
            return 0
        lax.fori_loop(0, C, zrow, 0)
        for j in range(RPT // C):
            pltpu.sync_copy(rows, acc.at[pl.ds(t * RPT + j * C, C)])
        plsc.subcore_barrier()

        def chunk(k, _):
            base = t * EPT + k * C
            pltpu.sync_copy(s_h.at[pl.ds(base, C)], sv)
            pltpu.sync_copy(r_h.at[pl.ds(base, C)], rv)

            def adj(i, _):
                sl = pl.ds(i * 16, 16)
                sv[sl] = sv[sl] * 2 + c
                return 0
            lax.fori_loop(0, C // 16, adj, 0)
            pltpu.async_copy(x2_h.at[sv], rows, sem).wait()
            if weighted:
                pltpu.sync_copy(w_h.at[pl.ds(base, C)], wv)

                def scale(i, _):
                    wvec = wv[pl.ds(i * 16, 16)]
                    for j in range(16):
                        we = wvec[j]
                        for k2 in range(8):
                            sl = pl.ds(k2 * 16, 16)
                            rows[i * 16 + j, sl] = rows[i * 16 + j, sl] * we
                    return 0
                lax.fori_loop(0, C // 16, scale, 0)
            pltpu.sync_copy(rows, acc.at[rv], add=True)
            return 0
        lax.fori_loop(0, EPT // C, chunk, 0)
        plsc.subcore_barrier()
        sl = pl.ds(t * RPT, RPT)

        @pl.when(c == 0)
        def _():
            pltpu.sync_copy(acc.at[sl], ya_h.at[sl])

        @pl.when(c == 1)
        def _():
            pltpu.sync_copy(acc.at[sl], yb_h.at[sl])

    return pl.kernel(
        body,
        out_type=[jax.ShapeDtypeStruct((NP, 128), jnp.float32)] * 2,
        mesh=mesh, scratch_types=scratch,
        compiler_params=pltpu.CompilerParams(needs_layout_passes=False))


def _sc_spmm(x, s_idx, r_idx, w=None):
    x2 = x.reshape(2 * N, 128)
    if w is None:
        w = jnp.zeros((8,), jnp.float32)
    ya, yb = _sc_spmm_call(w.shape[0] == E)(x2, s_idx, r_idx, w)
    return jnp.concatenate([ya[:N], yb[:N]], axis=1)


def _sc_gatw_call():
    mesh = plsc.VectorSubcoreMesh(**_MESH)
    HALF = EPT // 2
    scratch = [
        pltpu.VMEM((NP,), jnp.float32),
        pltpu.VMEM((NP,), jnp.float32),
        pltpu.VMEM((NP,), jnp.float32),
        pltpu.VMEM((EPT,), jnp.float32),
        pltpu.VMEM((C,), jnp.float32),
        pltpu.VMEM((C,), jnp.int32),
        pltpu.VMEM((C,), jnp.int32),
        pltpu.VMEM((C,), jnp.float32),
        pltpu.VMEM((RPT,), jnp.float32),
        pltpu.VMEM((RPT,), jnp.float32),
        pltpu.VMEM_SHARED((NT, NP), jnp.float32),
        pltpu.VMEM_SHARED((NP,), jnp.float32),
    ]

    def body(as_h, ar_h, b_h, s_h, r_h, w_h,
             asv, arv, den, exv, bb, sv, rv, wbuf, tmpv, comb,
             spm, spm_den):
        c = lax.axis_index("c")
        t = lax.axis_index("s")
        pltpu.sync_copy(as_h, asv)
        pltpu.sync_copy(ar_h, arv)
        _zero_vmem(den, NP)

        def chunk(k, _):
            base = t * EPT + k * C
            pltpu.sync_copy(s_h.at[pl.ds(base, C)], sv)
            pltpu.sync_copy(r_h.at[pl.ds(base, C)], rv)
            pltpu.sync_copy(b_h.at[pl.ds(base, C)], bb)

            def grp(i, _):
                sl = pl.ds(i * 16, 16)
                svec = sv[sl]
                rvec = rv[sl]
                sa = plsc.load_gather(asv, [svec])
                sr = plsc.load_gather(arv, [rvec])
                ex16 = jnp.exp(sa + sr + bb[sl])
                exv[pl.ds(k * C + i * 16, 16)] = ex16
                plsc.addupdate_scatter(den, [rvec], ex16)
                return 0
            lax.fori_loop(0, C // 16, grp, 0)
            return 0
        lax.fori_loop(0, EPT // C, chunk, 0)

        pltpu.sync_copy(den, spm.at[t])
        plsc.subcore_barrier()
        _zero_vmem(comb, RPT)
        for p in range(NT):
            pltpu.sync_copy(spm.at[p, pl.ds(t * RPT, RPT)], tmpv)

            def acc(i, _):
                sl = pl.ds(i * 16, 16)
                comb[sl] = comb[sl] + tmpv[sl]
                return 0
            lax.fori_loop(0, RPT // 16, acc, 0)
        pltpu.sync_copy(comb, spm_den.at[pl.ds(t * RPT, RPT)])
        plsc.subcore_barrier()
        pltpu.sync_copy(spm_den, den)

        def chunk2(k, _):
            off = c * HALF + k * C
            base = t * EPT + off
            pltpu.sync_copy(r_h.at[pl.ds(base, C)], rv)

            def grp2(i, _):
                sl = pl.ds(i * 16, 16)
                rvec = rv[sl]
                dv = plsc.load_gather(den, [rvec])
                ex16 = exv[pl.ds(off + i * 16, 16)]
                wbuf[sl] = ex16 / dv
                return 0
            lax.fori_loop(0, C // 16, grp2, 0)
            pltpu.sync_copy(wbuf, w_h.at[pl.ds(base, C)])
            return 0
        lax.fori_loop(0, HALF // C, chunk2, 0)

    return pl.kernel(
        body,
        out_type=jax.ShapeDtypeStruct((E,), jnp.float32),
        mesh=mesh, scratch_types=scratch,
        compiler_params=pltpu.CompilerParams(needs_layout_passes=False))


def _sc_gatw(a_s, a_r, base, s_idx, r_idx):
    asp = jnp.pad(a_s, (0, NP - N))
    arp = jnp.pad(a_r, (0, NP - N))
    return _sc_gatw_call()(asp, arp, base, s_idx, r_idx)


def _edge_base(edges, wl_e):
    wpad = jnp.zeros((16, 128), jnp.float32).at[:, 0].set(wl_e)
    out = _mm([(edges, wpad)], jnp.zeros((128,), jnp.float32), bm=4000)
    return out[:, 0]


def _mm2_body(x_ref, w_ref, b_ref, o_ref):
    acc = jax.lax.dot_general(
        x_ref[...], w_ref[0], (((1,), (0,)), ((), ())),
        precision=HI, preferred_element_type=jnp.float32)
    o_ref[0] = acc + b_ref[0]


def _mm2(x, w, b, bm):
    m = x.shape[0]
    k = x.shape[1]
    w2 = w.reshape(k, 2, 128).transpose(1, 0, 2)
    b2 = b.reshape(1, 2, 128).transpose(1, 0, 2)
    return pl.pallas_call(
        _mm2_body,
        grid=(m // bm, 2),
        in_specs=[pl.BlockSpec((bm, k), lambda i, q: (i, 0)),
                  pl.BlockSpec((1, k, 128), lambda i, q: (q, 0, 0)),
                  pl.BlockSpec((1, 1, 128), lambda i, q: (q, 0, 0))],
        out_specs=pl.BlockSpec((1, bm, 128), lambda i, q: (q, i, 0)),
        out_shape=jax.ShapeDtypeStruct((2, m, 128), jnp.float32),
    )(x, w2, b2)


def _sc_gnet_call():
    mesh = plsc.VectorSubcoreMesh(**_MESH)
    scratch = [
        pltpu.VMEM((C,), jnp.int32),
        pltpu.VMEM((C,), jnp.int32),
        pltpu.VMEM((C,), jnp.int32),
        pltpu.VMEM((C,), jnp.int32),
        pltpu.VMEM((C, 128), jnp.float32),
        pltpu.VMEM((C, 128), jnp.float32),
        pltpu.VMEM((C, 128), jnp.float32),
        pltpu.VMEM_SHARED((NP, 128), jnp.float32),
        pltpu.SemaphoreType.DMA,
        pltpu.SemaphoreType.DMA,
    ]

    def body(ps2_h, pr2_h, eq2_h, s_h, r_h, outs_h, outr_h,
             sv, rv, sg, rg, eqb, psb, prb, acc, sem1, sem2):
        c = lax.axis_index("c")
        t = lax.axis_index("s")
        for p in range(2):
            def zrow(e, _):
                for k in range(8):
                    eqb[e, pl.ds(k * 16, 16)] = jnp.zeros((16,), jnp.float32)
                return 0
            lax.fori_loop(0, C, zrow, 0)
            for jz in range(RPT // C):
                pltpu.sync_copy(eqb, acc.at[pl.ds(t * RPT + jz * C, C)])
            plsc.subcore_barrier()

            def chunk(k, _):
                base = t * EPT + k * C
                pltpu.sync_copy(s_h.at[pl.ds(base, C)], sv)
                pltpu.sync_copy(r_h.at[pl.ds(base, C)], rv)

                def adj(i, _):
                    sl = pl.ds(i * 16, 16)
                    sg[sl] = sv[sl] + c * N
                    rg[sl] = rv[sl] + c * N
                    return 0
                lax.fori_loop(0, C // 16, adj, 0)
                d1 = pltpu.async_copy(ps2_h.at[sg], psb, sem1)
                d2 = pltpu.async_copy(pr2_h.at[rg], prb, sem2)
                pltpu.sync_copy(eq2_h.at[pl.ds(c * E + base, C)], eqb)
                d1.wait()
                d2.wait()

                def comb(e, _):
                    for k2 in range(8):
                        sl = pl.ds(k2 * 16, 16)
                        v = eqb[e, sl] + psb[e, sl] + prb[e, sl]
                        eqb[e, sl] = jnp.maximum(v, 0.0)
                    return 0
                lax.fori_loop(0, C, comb, 0)
                pltpu.sync_copy(eqb, acc.at[sv if p == 0 else rv], add=True)
                return 0
            lax.fori_loop(0, EPT // C, chunk, 0)
            plsc.subcore_barrier()
            src = pl.ds(t * RPT, RPT)
            dst = pl.ds(c * NP + t * RPT, RPT)
            out_h = outs_h if p == 0 else outr_h
            pltpu.sync_copy(acc.at[src], out_h.at[dst])
            plsc.subcore_barrier()

    return pl.kernel(
        body,
        out_type=[jax.ShapeDtypeStruct((2 * NP, 128), jnp.float32)] * 2,
        mesh=mesh, scratch_types=scratch,
        compiler_params=pltpu.CompilerParams(needs_layout_passes=False))


def _sc_gnet(ps2, pr2, eq2, s_idx, r_idx):
    outs2, outr2 = _sc_gnet_call()(
        ps2.reshape(2 * N, 128), pr2.reshape(2 * N, 128),
        eq2.reshape(2 * E, 128), s_idx, r_idx)
    sa = outs2.reshape(2, NP, 128)[:, :N].transpose(1, 0, 2).reshape(N, 256)
    ra = outr2.reshape(2, NP, 128)[:, :N].transpose(1, 0, 2).reshape(N, 256)
    return sa, ra


def _sc_degrees_call():
    mesh = plsc.VectorSubcoreMesh(**_MESH)
    scratch = [
        pltpu.VMEM((NP,), jnp.float32),
        pltpu.VMEM((C,), jnp.int32),
        pltpu.VMEM((RPT,), jnp.float32),
        pltpu.VMEM((RPT,), jnp.float32),
        pltpu.VMEM_SHARED((NT, NP), jnp.float32),
    ]

    def body(s_h, r_h, sd_h, rd_h, part, iv, tmpv, comb, spm):
        c = lax.axis_index("c")
        t = lax.axis_index("s")

        def hist(idx_h, out_h):
            _zero_vmem(part, NP)
            ones16 = jnp.ones((16,), jnp.float32)

            def chunk(k, _):
                pltpu.sync_copy(idx_h.at[pl.ds(t * EPT + k * C, C)], iv)

                def grp(i, _):
                    plsc.addupdate_scatter(part, [iv[pl.ds(i * 16, 16)]],
                                           ones16)
                    return 0
                lax.fori_loop(0, C // 16, grp, 0)
                return 0
            lax.fori_loop(0, EPT // C, chunk, 0)
            pltpu.sync_copy(part, spm.at[t])
            plsc.subcore_barrier()
            _zero_vmem(comb, RPT)
            for p in range(NT):
                pltpu.sync_copy(spm.at[p, pl.ds(t * RPT, RPT)], tmpv)

                def acc(i, _):
                    sl = pl.ds(i * 16, 16)
                    comb[sl] = comb[sl] + tmpv[sl]
                    return 0
                lax.fori_loop(0, RPT // 16, acc, 0)
            pltpu.sync_copy(comb, out_h.at[pl.ds(t * RPT, RPT)])

        @pl.when(c == 0)
        def _():
            hist(s_h, sd_h)

        @pl.when(c == 1)
        def _():
            hist(r_h, rd_h)

    return pl.kernel(
        body,
        out_type=[jax.ShapeDtypeStruct((NP,), jnp.float32)] * 2,
        mesh=mesh, scratch_types=scratch,
        compiler_params=pltpu.CompilerParams(needs_layout_passes=False))


def _mm_body(nx, act, *refs):
    xs = refs[:nx]
    ws = refs[nx:2 * nx]
    b = refs[2 * nx]
    o = refs[2 * nx + 1]
    acc = jnp.zeros(o.shape, jnp.float32)
    for x, w in zip(xs, ws):
        acc = acc + jax.lax.dot_general(
            x[...], w[...], (((1,), (0,)), ((), ())),
            precision=HI, preferred_element_type=jnp.float32)
    acc = acc + b[...]
    if act == 'relu':
        acc = jnp.maximum(acc, 0.0)
    elif act == 'leaky':
        acc = jnp.where(acc >= 0, acc, 0.01 * acc)
    o[...] = acc


def _mm(xs_ws, b, act=None, bm=2000):
    xs = [x for x, _ in xs_ws]
    ws = [w for _, w in xs_ws]
    m = xs[0].shape[0]
    nout = ws[0].shape[1]
    grid = (m // bm,)
    in_specs = (
        [pl.BlockSpec((bm, x.shape[1]), lambda i: (i, 0)) for x in xs]
        + [pl.BlockSpec(w.shape, lambda i: (0, 0)) for w in ws]
        + [pl.BlockSpec((1, nout), lambda i: (0, 0))]
    )
    return pl.pallas_call(
        functools.partial(_mm_body, len(xs), act),
        grid=grid,
        in_specs=in_specs,
        out_specs=pl.BlockSpec((bm, nout), lambda i: (i, 0)),
        out_shape=jax.ShapeDtypeStruct((m, nout), jnp.float32),
    )(*xs, *ws, b.reshape(1, -1))


def _seg_sum(vals, idx, n):
    return jax.ops.segment_sum(vals, idx, num_segments=n)


def _seg_max(vals, idx, n):
    return jax.ops.segment_max(vals, idx, num_segments=n)


def _spmm(x, s_idx, r_idx, w=None):
    rows = x[s_idx]
    if w is not None:
        rows = rows * w[:, None]
    return _seg_sum(rows, r_idx, x.shape[0])


def _gat_weights(a_s, a_r, base, s_idx, r_idx):
    logits = a_s[s_idx] + a_r[r_idx] + base
    mx = _seg_max(logits, r_idx, N)
    e = jnp.exp(logits - mx[r_idx])
    den = _seg_sum(e, r_idx, N)
    return e / den[r_idx]


def kernel(nodes, edges, senders, receivers, globals_,
           W_gcn1, b_gcn1, W_gcn2, b_gcn2,
           Wq1, bq1, Wl1, bl1,
           Wq2, bq2, Wl2, bl2, Wn2, bn2,
           We, be, Wn, bn, Wg, bg):
    s_idx = senders.astype(jnp.int32)
    r_idx = receivers.astype(jnp.int32)
    sdeg, rdeg = _sc_degrees_call()(s_idx, r_idx)
    ssc = jax.lax.rsqrt(sdeg[:N, None] + 1.0)
    rsc = jax.lax.rsqrt(rdeg[:N, None] + 1.0)

    h = _mm([(nodes, W_gcn1)], b_gcn1, act='relu') * ssc
    h = (h + _sc_spmm(h, s_idx, r_idx)) * rsc
    h = _mm([(h, W_gcn2)], b_gcn2, act='relu') * ssc
    h = (h + _sc_spmm(h, s_idx, r_idx)) * rsc

    q = _mm([(h, Wq1)], bq1)
    wl_s, wl_r, wl_e = Wl1[:256, 0], Wl1[256:512, 0], Wl1[512:, 0]
    a_s = q @ wl_s + bl1[0]
    a_r = q @ wl_r
    w1 = _sc_gatw(a_s, a_r, _edge_base(edges, wl_e), s_idx, r_idx)
    h = _sc_spmm(q, s_idx, r_idx, w1)
    h = jnp.where(h >= 0, h, 0.01 * h)

    q = _mm([(h, Wq2)], bq2)
    wl_s, wl_r, wl_e = Wl2[:256, 0], Wl2[256:512, 0], Wl2[512:, 0]
    a_s = q @ wl_s + bl2[0]
    a_r = q @ wl_r
    w2 = _sc_gatw(a_s, a_r, _edge_base(edges, wl_e), s_idx, r_idx)
    agg = _sc_spmm(q, s_idx, r_idx, w2)
    h = _mm([(agg, Wn2)], bn2)

    We_e, We_s, We_r = We[:16], We[16:272], We[272:528]
    z256 = jnp.zeros((256,), jnp.float32)
    ps2 = _mm2(h, We_s, z256, bm=2000)
    pr2 = _mm2(h, We_r, z256, bm=2000)
    eq2 = _mm2(edges, We_e, be, bm=4000)
    sent_agg, recv_agg = _sc_gnet(ps2, pr2, eq2, s_idx, r_idx)

    Wn_h, Wn_s, Wn_r = Wn[:256], Wn[256:512], Wn[512:768]
    new_nodes = _mm([(h, Wn_h), (sent_agg, Wn_s), (recv_agg, Wn_r)], bn,
                    act='relu')

    node_attr = jnp.sum(new_nodes, axis=0, keepdims=True)
    edge_attr = jnp.sum(sent_agg, axis=0, keepdims=True)
    g = jnp.zeros((1, 64), jnp.float32)
    feats = jnp.concatenate([node_attr, edge_attr, g], axis=1)
    logits = jnp.maximum(feats @ Wg + bg, 0.0)
    return logits


# speedup vs baseline: 6.7883x; 1.5304x over previous
import functools
import jax
import jax.numpy as jnp
from jax import lax
from jax.experimental import pallas as pl
from jax.experimental.pallas import tpu as pltpu
from jax.experimental.pallas import tpu_sc as plsc

N = 10000
E = 320000
NP = 10240
NT = 16
C = 80
EPT = E // NT
SEG = 4000
RPT = NP // NT
HI = jax.lax.Precision.HIGHEST
_MESH = dict(core_axis_name="c", subcore_axis_name="s",
             num_cores=2, num_subcores=NT)


def _zero_vmem(ref, n):
    z = jnp.zeros((16,), jnp.float32)

    def zr(i, _):
        ref[pl.ds(i * 16, 16)] = z
        return 0
    lax.fori_loop(0, n // 16, zr, 0)


def _sc_spmm_call(weighted):
    mesh = plsc.VectorSubcoreMesh(**_MESH)
    scratch = [
        pltpu.VMEM((SEG,), jnp.int32),
        pltpu.VMEM((SEG,), jnp.int32),
        pltpu.VMEM((SEG,), jnp.float32) if weighted else None,
        pltpu.VMEM((C,), jnp.int32),
        pltpu.VMEM((C,), jnp.int32),
        pltpu.VMEM((C, 128), jnp.float32),
        pltpu.VMEM_SHARED((NP, 128), jnp.float32),
        pltpu.SemaphoreType.DMA,
    ]
    scratch = [s for s in scratch if s is not None]

    def body(x2_h, s_h, r_h, w_h, ya_h, yb_h, *rest):
        if weighted:
            sia, ria, wa, sg, rc, rows, acc, sem = rest
        else:
            sia, ria, sg, rc, rows, acc, sem = rest
        c = lax.axis_index("c")
        t = lax.axis_index("s")

        def zrow(e, _):
            for k in range(8):
                rows[e, pl.ds(k * 16, 16)] = jnp.zeros((16,), jnp.float32)
            return 0
        lax.fori_loop(0, C, zrow, 0)
        for j in range(RPT // C):
            pltpu.sync_copy(rows, acc.at[pl.ds(t * RPT + j * C, C)])
        plsc.subcore_barrier()

        def seg(g, _):
            sbase = t * EPT + g * SEG
            pltpu.sync_copy(s_h.at[pl.ds(sbase, SEG)], sia)
            pltpu.sync_copy(r_h.at[pl.ds(sbase, SEG)], ria)
            if weighted:
                pltpu.sync_copy(w_h.at[pl.ds(sbase, SEG)], wa)

            def chunk(k, _):
                def adj(i, _):
                    sl = pl.ds(i * 16, 16)
                    src = pl.ds(k * C + i * 16, 16)
                    sg[sl] = sia[src] * 2 + c
                    rc[sl] = ria[src]
                    return 0
                lax.fori_loop(0, C // 16, adj, 0)
                pltpu.async_copy(x2_h.at[sg], rows, sem).wait()
                if weighted:
                    def scale(i, _):
                        wvec = wa[pl.ds(k * C + i * 16, 16)]
                        for j in range(16):
                            we = wvec[j]
                            for k2 in range(8):
                                sl = pl.ds(k2 * 16, 16)
                                rows[i * 16 + j, sl] = (
                                    rows[i * 16 + j, sl] * we)
                        return 0
                    lax.fori_loop(0, C // 16, scale, 0)
                pltpu.sync_copy(rows, acc.at[rc], add=True)
                return 0
            lax.fori_loop(0, SEG // C, chunk, 0)
            return 0
        lax.fori_loop(0, EPT // SEG, seg, 0)
        plsc.subcore_barrier()
        sl = pl.ds(t * RPT, RPT)

        @pl.when(c == 0)
        def _():
            pltpu.sync_copy(acc.at[sl], ya_h.at[sl])

        @pl.when(c == 1)
        def _():
            pltpu.sync_copy(acc.at[sl], yb_h.at[sl])

    return pl.kernel(
        body,
        out_type=[jax.ShapeDtypeStruct((NP, 128), jnp.float32)] * 2,
        mesh=mesh, scratch_types=scratch,
        compiler_params=pltpu.CompilerParams(needs_layout_passes=False))


def _sc_spmm(x, s_idx, r_idx, w=None):
    x2 = x.reshape(2 * N, 128)
    if w is None:
        w = jnp.zeros((8,), jnp.float32)
    ya, yb = _sc_spmm_call(w.shape[0] == E)(x2, s_idx, r_idx, w)
    return jnp.concatenate([ya[:N], yb[:N]], axis=1)


def _sc_gatw_call():
    mesh = plsc.VectorSubcoreMesh(**_MESH)
    HALF = EPT // 2
    scratch = [
        pltpu.VMEM((NP,), jnp.float32),
        pltpu.VMEM((NP,), jnp.float32),
        pltpu.VMEM((NP,), jnp.float32),
        pltpu.VMEM((EPT,), jnp.float32),
        pltpu.VMEM((EPT,), jnp.float32),
        pltpu.VMEM((EPT,), jnp.int32),
        pltpu.VMEM((EPT,), jnp.int32),
        pltpu.VMEM((C,), jnp.float32),
        pltpu.VMEM((RPT,), jnp.float32),
        pltpu.VMEM((RPT,), jnp.float32),
        pltpu.VMEM_SHARED((NT, NP), jnp.float32),
        pltpu.VMEM_SHARED((NP,), jnp.float32),
    ]

    def body(as_h, ar_h, b_h, s_h, r_h, w_h,
             asv, arv, den, exv, bba, sia, ria, wbuf, tmpv, comb,
             spm, spm_den):
        c = lax.axis_index("c")
        t = lax.axis_index("s")
        pltpu.sync_copy(as_h, asv)
        pltpu.sync_copy(ar_h, arv)
        pltpu.sync_copy(s_h.at[pl.ds(t * EPT, EPT)], sia)
        pltpu.sync_copy(r_h.at[pl.ds(t * EPT, EPT)], ria)
        pltpu.sync_copy(b_h.at[pl.ds(t * EPT, EPT)], bba)
        _zero_vmem(den, NP)

        def grp(i, _):
            sl = pl.ds(i * 16, 16)
            svec = sia[sl]
            rvec = ria[sl]
            sa = plsc.load_gather(asv, [svec])
            sr = plsc.load_gather(arv, [rvec])
            ex16 = jnp.exp(sa + sr + bba[sl])
            exv[sl] = ex16
            plsc.addupdate_scatter(den, [rvec], ex16)
            return 0
        lax.fori_loop(0, EPT // 16, grp, 0)

        pltpu.sync_copy(den, spm.at[t])
        plsc.subcore_barrier()
        _zero_vmem(comb, RPT)
        for p in range(NT):
            pltpu.sync_copy(spm.at[p, pl.ds(t * RPT, RPT)], tmpv)

            def acc(i, _):
                sl = pl.ds(i * 16, 16)
                comb[sl] = comb[sl] + tmpv[sl]
                return 0
            lax.fori_loop(0, RPT // 16, acc, 0)
        pltpu.sync_copy(comb, spm_den.at[pl.ds(t * RPT, RPT)])
        plsc.subcore_barrier()
        pltpu.sync_copy(spm_den, den)

        def chunk2(k, _):
            off = c * HALF + k * C

            def grp2(i, _):
                sl = pl.ds(i * 16, 16)
                src = pl.ds(off + i * 16, 16)
                dv = plsc.load_gather(den, [ria[src]])
                wbuf[sl] = exv[src] / dv
                return 0
            lax.fori_loop(0, C // 16, grp2, 0)
            pltpu.sync_copy(wbuf, w_h.at[pl.ds(t * EPT + off, C)])
            return 0
        lax.fori_loop(0, HALF // C, chunk2, 0)

    return pl.kernel(
        body,
        out_type=jax.ShapeDtypeStruct((E,), jnp.float32),
        mesh=mesh, scratch_types=scratch,
        compiler_params=pltpu.CompilerParams(needs_layout_passes=False))


def _sc_gatw(a_s, a_r, base, s_idx, r_idx):
    asp = jnp.pad(a_s, (0, NP - N))
    arp = jnp.pad(a_r, (0, NP - N))
    return _sc_gatw_call()(asp, arp, base, s_idx, r_idx)


def _edge_base(edges, wl_e):
    wpad = jnp.zeros((16, 128), jnp.float32).at[:, 0].set(wl_e)
    out = _mm([(edges, wpad)], jnp.zeros((128,), jnp.float32), bm=4000)
    return out[:, 0]


def _mm2_body(x_ref, w_ref, b_ref, o_ref):
    acc = jax.lax.dot_general(
        x_ref[...], w_ref[0], (((1,), (0,)), ((), ())),
        precision=HI, preferred_element_type=jnp.float32)
    o_ref[0] = acc + b_ref[0]


def _mm2(x, w, b, bm):
    m = x.shape[0]
    k = x.shape[1]
    w2 = w.reshape(k, 2, 128).transpose(1, 0, 2)
    b2 = b.reshape(1, 2, 128).transpose(1, 0, 2)
    return pl.pallas_call(
        _mm2_body,
        grid=(m // bm, 2),
        in_specs=[pl.BlockSpec((bm, k), lambda i, q: (i, 0)),
                  pl.BlockSpec((1, k, 128), lambda i, q: (q, 0, 0)),
                  pl.BlockSpec((1, 1, 128), lambda i, q: (q, 0, 0))],
        out_specs=pl.BlockSpec((1, bm, 128), lambda i, q: (q, i, 0)),
        out_shape=jax.ShapeDtypeStruct((2, m, 128), jnp.float32),
    )(x, w2, b2)


def _sc_gnet_call():
    mesh = plsc.VectorSubcoreMesh(**_MESH)
    scratch = [
        pltpu.VMEM((SEG,), jnp.int32),
        pltpu.VMEM((SEG,), jnp.int32),
        pltpu.VMEM((C,), jnp.int32),
        pltpu.VMEM((C,), jnp.int32),
        pltpu.VMEM((C,), jnp.int32),
        pltpu.VMEM((C, 128), jnp.float32),
        pltpu.VMEM((C, 128), jnp.float32),
        pltpu.VMEM((C, 128), jnp.float32),
        pltpu.VMEM_SHARED((NP, 128), jnp.float32),
        pltpu.SemaphoreType.DMA,
        pltpu.SemaphoreType.DMA,
    ]

    def body(ps2_h, pr2_h, eq2_h, s_h, r_h, outs_h, outr_h,
             sia, ria, sg, rg, sci, eqb, psb, prb, acc, sem1, sem2):
        c = lax.axis_index("c")
        t = lax.axis_index("s")
        for p in range(2):
            def zrow(e, _):
                for k in range(8):
                    eqb[e, pl.ds(k * 16, 16)] = jnp.zeros((16,), jnp.float32)
                return 0
            lax.fori_loop(0, C, zrow, 0)
            for jz in range(RPT // C):
                pltpu.sync_copy(eqb, acc.at[pl.ds(t * RPT + jz * C, C)])
            plsc.subcore_barrier()

            def seg(g, _):
                sbase = t * EPT + g * SEG
                pltpu.sync_copy(s_h.at[pl.ds(sbase, SEG)], sia)
                pltpu.sync_copy(r_h.at[pl.ds(sbase, SEG)], ria)
                sci_src = sia if p == 0 else ria

                def chunk(k, _):
                    def adj(i, _):
                        sl = pl.ds(i * 16, 16)
                        src = pl.ds(k * C + i * 16, 16)
                        sg[sl] = sia[src] + c * N
                        rg[sl] = ria[src] + c * N
                        sci[sl] = sci_src[src]
                        return 0
                    lax.fori_loop(0, C // 16, adj, 0)
                    d1 = pltpu.async_copy(ps2_h.at[sg], psb, sem1)
                    d2 = pltpu.async_copy(pr2_h.at[rg], prb, sem2)
                    pltpu.sync_copy(
                        eq2_h.at[pl.ds(c * E + sbase + k * C, C)], eqb)
                    d1.wait()
                    d2.wait()

                    def comb(e, _):
                        for k2 in range(8):
                            sl = pl.ds(k2 * 16, 16)
                            v = eqb[e, sl] + psb[e, sl] + prb[e, sl]
                            eqb[e, sl] = jnp.maximum(v, 0.0)
                        return 0
                    lax.fori_loop(0, C, comb, 0)
                    pltpu.sync_copy(eqb, acc.at[sci], add=True)
                    return 0
                lax.fori_loop(0, SEG // C, chunk, 0)
                return 0
            lax.fori_loop(0, EPT // SEG, seg, 0)
            plsc.subcore_barrier()
            src = pl.ds(t * RPT, RPT)
            dst = pl.ds(c * NP + t * RPT, RPT)
            out_h = outs_h if p == 0 else outr_h
            pltpu.sync_copy(acc.at[src], out_h.at[dst])
            plsc.subcore_barrier()

    return pl.kernel(
        body,
        out_type=[jax.ShapeDtypeStruct((2 * NP, 128), jnp.float32)] * 2,
        mesh=mesh, scratch_types=scratch,
        compiler_params=pltpu.CompilerParams(needs_layout_passes=False))


def _sc_gnet(ps2, pr2, eq2, s_idx, r_idx):
    outs2, outr2 = _sc_gnet_call()(
        ps2.reshape(2 * N, 128), pr2.reshape(2 * N, 128),
        eq2.reshape(2 * E, 128), s_idx, r_idx)
    sa = outs2.reshape(2, NP, 128)[:, :N].transpose(1, 0, 2).reshape(N, 256)
    ra = outr2.reshape(2, NP, 128)[:, :N].transpose(1, 0, 2).reshape(N, 256)
    return sa, ra


def _sc_degrees_call():
    mesh = plsc.VectorSubcoreMesh(**_MESH)
    scratch = [
        pltpu.VMEM((NP,), jnp.float32),
        pltpu.VMEM((EPT,), jnp.int32),
        pltpu.VMEM((RPT,), jnp.float32),
        pltpu.VMEM((RPT,), jnp.float32),
        pltpu.VMEM_SHARED((NT, NP), jnp.float32),
    ]

    def body(s_h, r_h, sd_h, rd_h, part, iv, tmpv, comb, spm):
        c = lax.axis_index("c")
        t = lax.axis_index("s")

        def hist(idx_h, out_h):
            _zero_vmem(part, NP)
            ones16 = jnp.ones((16,), jnp.float32)
            pltpu.sync_copy(idx_h.at[pl.ds(t * EPT, EPT)], iv)

            def grp(i, _):
                plsc.addupdate_scatter(part, [iv[pl.ds(i * 16, 16)]],
                                       ones16)
                return 0
            lax.fori_loop(0, EPT // 16, grp, 0)
            pltpu.sync_copy(part, spm.at[t])
            plsc.subcore_barrier()
            _zero_vmem(comb, RPT)
            for p in range(NT):
                pltpu.sync_copy(spm.at[p, pl.ds(t * RPT, RPT)], tmpv)

                def acc(i, _):
                    sl = pl.ds(i * 16, 16)
                    comb[sl] = comb[sl] + tmpv[sl]
                    return 0
                lax.fori_loop(0, RPT // 16, acc, 0)
            pltpu.sync_copy(comb, out_h.at[pl.ds(t * RPT, RPT)])

        @pl.when(c == 0)
        def _():
            hist(s_h, sd_h)

        @pl.when(c == 1)
        def _():
            hist(r_h, rd_h)

    return pl.kernel(
        body,
        out_type=[jax.ShapeDtypeStruct((NP,), jnp.float32)] * 2,
        mesh=mesh, scratch_types=scratch,
        compiler_params=pltpu.CompilerParams(needs_layout_passes=False))


def _mm_body(nx, act, *refs):
    xs = refs[:nx]
    ws = refs[nx:2 * nx]
    b = refs[2 * nx]
    o = refs[2 * nx + 1]
    acc = jnp.zeros(o.shape, jnp.float32)
    for x, w in zip(xs, ws):
        acc = acc + jax.lax.dot_general(
            x[...], w[...], (((1,), (0,)), ((), ())),
            precision=HI, preferred_element_type=jnp.float32)
    acc = acc + b[...]
    if act == 'relu':
        acc = jnp.maximum(acc, 0.0)
    elif act == 'leaky':
        acc = jnp.where(acc >= 0, acc, 0.01 * acc)
    o[...] = acc


def _mm(xs_ws, b, act=None, bm=2000):
    xs = [x for x, _ in xs_ws]
    ws = [w for _, w in xs_ws]
    m = xs[0].shape[0]
    nout = ws[0].shape[1]
    grid = (m // bm,)
    in_specs = (
        [pl.BlockSpec((bm, x.shape[1]), lambda i: (i, 0)) for x in xs]
        + [pl.BlockSpec(w.shape, lambda i: (0, 0)) for w in ws]
        + [pl.BlockSpec((1, nout), lambda i: (0, 0))]
    )
    return pl.pallas_call(
        functools.partial(_mm_body, len(xs), act),
        grid=grid,
        in_specs=in_specs,
        out_specs=pl.BlockSpec((bm, nout), lambda i: (i, 0)),
        out_shape=jax.ShapeDtypeStruct((m, nout), jnp.float32),
    )(*xs, *ws, b.reshape(1, -1))


def _seg_sum(vals, idx, n):
    return jax.ops.segment_sum(vals, idx, num_segments=n)


def _seg_max(vals, idx, n):
    return jax.ops.segment_max(vals, idx, num_segments=n)


def _spmm(x, s_idx, r_idx, w=None):
    rows = x[s_idx]
    if w is not None:
        rows = rows * w[:, None]
    return _seg_sum(rows, r_idx, x.shape[0])


def _gat_weights(a_s, a_r, base, s_idx, r_idx):
    logits = a_s[s_idx] + a_r[r_idx] + base
    mx = _seg_max(logits, r_idx, N)
    e = jnp.exp(logits - mx[r_idx])
    den = _seg_sum(e, r_idx, N)
    return e / den[r_idx]


def kernel(nodes, edges, senders, receivers, globals_,
           W_gcn1, b_gcn1, W_gcn2, b_gcn2,
           Wq1, bq1, Wl1, bl1,
           Wq2, bq2, Wl2, bl2, Wn2, bn2,
           We, be, Wn, bn, Wg, bg):
    s_idx = senders.astype(jnp.int32)
    r_idx = receivers.astype(jnp.int32)
    sdeg, rdeg = _sc_degrees_call()(s_idx, r_idx)
    ssc = jax.lax.rsqrt(sdeg[:N, None] + 1.0)
    rsc = jax.lax.rsqrt(rdeg[:N, None] + 1.0)

    h = _mm([(nodes, W_gcn1)], b_gcn1, act='relu') * ssc
    h = (h + _sc_spmm(h, s_idx, r_idx)) * rsc
    h = _mm([(h, W_gcn2)], b_gcn2, act='relu') * ssc
    h = (h + _sc_spmm(h, s_idx, r_idx)) * rsc

    q = _mm([(h, Wq1)], bq1)
    wl_s, wl_r, wl_e = Wl1[:256, 0], Wl1[256:512, 0], Wl1[512:, 0]
    a_s = q @ wl_s + bl1[0]
    a_r = q @ wl_r
    w1 = _sc_gatw(a_s, a_r, _edge_base(edges, wl_e), s_idx, r_idx)
    h = _sc_spmm(q, s_idx, r_idx, w1)
    h = jnp.where(h >= 0, h, 0.01 * h)

    q = _mm([(h, Wq2)], bq2)
    wl_s, wl_r, wl_e = Wl2[:256, 0], Wl2[256:512, 0], Wl2[512:, 0]
    a_s = q @ wl_s + bl2[0]
    a_r = q @ wl_r
    w2 = _sc_gatw(a_s, a_r, _edge_base(edges, wl_e), s_idx, r_idx)
    agg = _sc_spmm(q, s_idx, r_idx, w2)
    h = _mm([(agg, Wn2)], bn2)

    We_e, We_s, We_r = We[:16], We[16:272], We[272:528]
    z256 = jnp.zeros((256,), jnp.float32)
    ps2 = _mm2(h, We_s, z256, bm=2000)
    pr2 = _mm2(h, We_r, z256, bm=2000)
    eq2 = _mm2(edges, We_e, be, bm=4000)
    sent_agg, recv_agg = _sc_gnet(ps2, pr2, eq2, s_idx, r_idx)

    Wn_h, Wn_s, Wn_r = Wn[:256], Wn[256:512], Wn[512:768]
    new_nodes = _mm([(h, Wn_h), (sent_agg, Wn_s), (recv_agg, Wn_r)], bn,
                    act='relu')

    node_attr = jnp.sum(new_nodes, axis=0, keepdims=True)
    edge_attr = jnp.sum(sent_agg, axis=0, keepdims=True)
    g = jnp.zeros((1, 64), jnp.float32)
    feats = jnp.concatenate([node_attr, edge_attr, g], axis=1)
    logits = jnp.maximum(feats @ Wg + bg, 0.0)
    return logits


# speedup vs baseline: 7.5734x; 1.1157x over previous
import functools
import jax
import jax.numpy as jnp
from jax import lax
from jax.experimental import pallas as pl
from jax.experimental.pallas import tpu as pltpu
from jax.experimental.pallas import tpu_sc as plsc

N = 10000
E = 320000
NP = 10240
NT = 16
C = 80
EPT = E // NT
SEG = 4000
RPT = NP // NT
HI = jax.lax.Precision.HIGHEST
_MESH = dict(core_axis_name="c", subcore_axis_name="s",
             num_cores=2, num_subcores=NT)


def _zero_vmem(ref, n):
    z = jnp.zeros((16,), jnp.float32)

    def zr(i, _):
        ref[pl.ds(i * 16, 16)] = z
        return 0
    lax.fori_loop(0, n // 16, zr, 0)


def _sc_spmm_call(weighted):
    mesh = plsc.VectorSubcoreMesh(**_MESH)
    scratch = [
        pltpu.VMEM((SEG,), jnp.int32),
        pltpu.VMEM((SEG,), jnp.int32),
        pltpu.VMEM((SEG,), jnp.float32) if weighted else None,
        pltpu.VMEM((C,), jnp.int32),
        pltpu.VMEM((C,), jnp.int32),
        pltpu.VMEM((C,), jnp.int32),
        pltpu.VMEM((C,), jnp.int32),
        pltpu.VMEM((C, 128), jnp.float32),
        pltpu.VMEM((C, 128), jnp.float32),
        pltpu.VMEM_SHARED((NP, 128), jnp.float32),
        pltpu.SemaphoreType.DMA,
        pltpu.SemaphoreType.DMA,
    ]
    scratch = [s for s in scratch if s is not None]

    def body(x2_h, s_h, r_h, w_h, ya_h, yb_h, *rest):
        if weighted:
            sia, ria, wa, sg0, sg1, rc0, rc1, rows0, rows1, acc, s0, s1 = rest
        else:
            sia, ria, sg0, sg1, rc0, rc1, rows0, rows1, acc, s0, s1 = rest
        c = lax.axis_index("c")
        t = lax.axis_index("s")

        def zrow(e, _):
            for k in range(8):
                rows0[e, pl.ds(k * 16, 16)] = jnp.zeros((16,), jnp.float32)
            return 0
        lax.fori_loop(0, C, zrow, 0)
        for j in range(RPT // C):
            pltpu.sync_copy(rows0, acc.at[pl.ds(t * RPT + j * C, C)])
        plsc.subcore_barrier()

        def seg(g, _):
            sbase = t * EPT + g * SEG
            pltpu.sync_copy(s_h.at[pl.ds(sbase, SEG)], sia)
            pltpu.sync_copy(r_h.at[pl.ds(sbase, SEG)], ria)
            if weighted:
                pltpu.sync_copy(w_h.at[pl.ds(sbase, SEG)], wa)

            def half_chunk(k, sg, rc, rows, sem):
                def adj(i, _):
                    sl = pl.ds(i * 16, 16)
                    src = pl.ds(k * C + i * 16, 16)
                    sg[sl] = sia[src] * 2 + c
                    rc[sl] = ria[src]
                    return 0
                lax.fori_loop(0, C // 16, adj, 0)
                return pltpu.async_copy(x2_h.at[sg], rows, sem)

            def drain(k, rc, rows):
                if weighted:
                    def scale(i, _):
                        wvec = wa[pl.ds(k * C + i * 16, 16)]
                        for j in range(16):
                            we = wvec[j]
                            for k2 in range(8):
                                sl = pl.ds(k2 * 16, 16)
                                rows[i * 16 + j, sl] = (
                                    rows[i * 16 + j, sl] * we)
                        return 0
                    lax.fori_loop(0, C // 16, scale, 0)
                pltpu.sync_copy(rows, acc.at[rc], add=True)

            def pair(p, _):
                k0 = 2 * p
                d0 = half_chunk(k0, sg0, rc0, rows0, s0)
                d1 = half_chunk(k0 + 1, sg1, rc1, rows1, s1)
                d0.wait()
                drain(k0, rc0, rows0)
                d1.wait()
                drain(k0 + 1, rc1, rows1)
                return 0
            lax.fori_loop(0, SEG // C // 2, pair, 0)
            return 0
        lax.fori_loop(0, EPT // SEG, seg, 0)
        plsc.subcore_barrier()
        sl = pl.ds(t * RPT, RPT)

        @pl.when(c == 0)
        def _():
            pltpu.sync_copy(acc.at[sl], ya_h.at[sl])

        @pl.when(c == 1)
        def _():
            pltpu.sync_copy(acc.at[sl], yb_h.at[sl])

    return pl.kernel(
        body,
        out_type=[jax.ShapeDtypeStruct((NP, 128), jnp.float32)] * 2,
        mesh=mesh, scratch_types=scratch,
        compiler_params=pltpu.CompilerParams(needs_layout_passes=False))


def _sc_spmm(x, s_idx, r_idx, w=None):
    x2 = x.reshape(2 * N, 128)
    if w is None:
        w = jnp.zeros((8,), jnp.float32)
    ya, yb = _sc_spmm_call(w.shape[0] == E)(x2, s_idx, r_idx, w)
    return jnp.concatenate([ya[:N], yb[:N]], axis=1)


def _sc_gatw_call():
    mesh = plsc.VectorSubcoreMesh(**_MESH)
    HALF = EPT // 2
    scratch = [
        pltpu.VMEM((NP,), jnp.float32),
        pltpu.VMEM((NP,), jnp.float32),
        pltpu.VMEM((NP,), jnp.float32),
        pltpu.VMEM((EPT,), jnp.float32),
        pltpu.VMEM((EPT,), jnp.float32),
        pltpu.VMEM((EPT,), jnp.int32),
        pltpu.VMEM((EPT,), jnp.int32),
        pltpu.VMEM((C,), jnp.float32),
        pltpu.VMEM((RPT,), jnp.float32),
        pltpu.VMEM((RPT,), jnp.float32),
        pltpu.VMEM_SHARED((NT, NP), jnp.float32),
        pltpu.VMEM_SHARED((NP,), jnp.float32),
    ]

    def body(as_h, ar_h, b_h, s_h, r_h, w_h,
             asv, arv, den, exv, bba, sia, ria, wbuf, tmpv, comb,
             spm, spm_den):
        c = lax.axis_index("c")
        t = lax.axis_index("s")
        pltpu.sync_copy(as_h, asv)
        pltpu.sync_copy(ar_h, arv)
        pltpu.sync_copy(s_h.at[pl.ds(t * EPT, EPT)], sia)
        pltpu.sync_copy(r_h.at[pl.ds(t * EPT, EPT)], ria)
        pltpu.sync_copy(b_h.at[pl.ds(t * EPT, EPT)], bba)
        _zero_vmem(den, NP)

        def grp(i, _):
            sl = pl.ds(i * 16, 16)
            svec = sia[sl]
            rvec = ria[sl]
            sa = plsc.load_gather(asv, [svec])
            sr = plsc.load_gather(arv, [rvec])
            ex16 = jnp.exp(sa + sr + bba[sl])
            exv[sl] = ex16
            plsc.addupdate_scatter(den, [rvec], ex16)
            return 0
        lax.fori_loop(0, EPT // 16, grp, 0)

        pltpu.sync_copy(den, spm.at[t])
        plsc.subcore_barrier()
        _zero_vmem(comb, RPT)
        for p in range(NT):
            pltpu.sync_copy(spm.at[p, pl.ds(t * RPT, RPT)], tmpv)

            def acc(i, _):
                sl = pl.ds(i * 16, 16)
                comb[sl] = comb[sl] + tmpv[sl]
                return 0
            lax.fori_loop(0, RPT // 16, acc, 0)
        pltpu.sync_copy(comb, spm_den.at[pl.ds(t * RPT, RPT)])
        plsc.subcore_barrier()
        pltpu.sync_copy(spm_den, den)

        def chunk2(k, _):
            off = c * HALF + k * C

            def grp2(i, _):
                sl = pl.ds(i * 16, 16)
                src = pl.ds(off + i * 16, 16)
                dv = plsc.load_gather(den, [ria[src]])
                wbuf[sl] = exv[src] / dv
                return 0
            lax.fori_loop(0, C // 16, grp2, 0)
            pltpu.sync_copy(wbuf, w_h.at[pl.ds(t * EPT + off, C)])
            return 0
        lax.fori_loop(0, HALF // C, chunk2, 0)

    return pl.kernel(
        body,
        out_type=jax.ShapeDtypeStruct((E,), jnp.float32),
        mesh=mesh, scratch_types=scratch,
        compiler_params=pltpu.CompilerParams(needs_layout_passes=False))


def _sc_gatw(a_s, a_r, base, s_idx, r_idx):
    asp = jnp.pad(a_s, (0, NP - N))
    arp = jnp.pad(a_r, (0, NP - N))
    return _sc_gatw_call()(asp, arp, base, s_idx, r_idx)


def _edge_base(edges, wl_e):
    wpad = jnp.zeros((16, 128), jnp.float32).at[:, 0].set(wl_e)
    out = _mm([(edges, wpad)], jnp.zeros((128,), jnp.float32), bm=4000)
    return out[:, 0]


def _mm2_body(x_ref, w_ref, b_ref, o_ref):
    acc = jax.lax.dot_general(
        x_ref[...], w_ref[0], (((1,), (0,)), ((), ())),
        precision=HI, preferred_element_type=jnp.float32)
    o_ref[0] = acc + b_ref[0]


def _mm2(x, w, b, bm):
    m = x.shape[0]
    k = x.shape[1]
    w2 = w.reshape(k, 2, 128).transpose(1, 0, 2)
    b2 = b.reshape(1, 2, 128).transpose(1, 0, 2)
    return pl.pallas_call(
        _mm2_body,
        grid=(m // bm, 2),
        in_specs=[pl.BlockSpec((bm, k), lambda i, q: (i, 0)),
                  pl.BlockSpec((1, k, 128), lambda i, q: (q, 0, 0)),
                  pl.BlockSpec((1, 1, 128), lambda i, q: (q, 0, 0))],
        out_specs=pl.BlockSpec((1, bm, 128), lambda i, q: (q, i, 0)),
        out_shape=jax.ShapeDtypeStruct((2, m, 128), jnp.float32),
    )(x, w2, b2)


def _sc_gnet_call():
    mesh = plsc.VectorSubcoreMesh(**_MESH)
    scratch = [
        pltpu.VMEM((SEG,), jnp.int32),
        pltpu.VMEM((SEG,), jnp.int32),
        pltpu.VMEM((C,), jnp.int32),
        pltpu.VMEM((C,), jnp.int32),
        pltpu.VMEM((C,), jnp.int32),
        pltpu.VMEM((C, 128), jnp.float32),
        pltpu.VMEM((C, 128), jnp.float32),
        pltpu.VMEM((C, 128), jnp.float32),
        pltpu.VMEM_SHARED((NP, 128), jnp.float32),
        pltpu.SemaphoreType.DMA,
        pltpu.SemaphoreType.DMA,
    ]

    def body(ps2_h, pr2_h, eq2_h, s_h, r_h, outs_h, outr_h,
             sia, ria, sg, rg, sci, eqb, psb, prb, acc, sem1, sem2):
        c = lax.axis_index("c")
        t = lax.axis_index("s")
        for p in range(2):
            def zrow(e, _):
                for k in range(8):
                    eqb[e, pl.ds(k * 16, 16)] = jnp.zeros((16,), jnp.float32)
                return 0
            lax.fori_loop(0, C, zrow, 0)
            for jz in range(RPT // C):
                pltpu.sync_copy(eqb, acc.at[pl.ds(t * RPT + jz * C, C)])
            plsc.subcore_barrier()

            def seg(g, _):
                sbase = t * EPT + g * SEG
                pltpu.sync_copy(s_h.at[pl.ds(sbase, SEG)], sia)
                pltpu.sync_copy(r_h.at[pl.ds(sbase, SEG)], ria)
                sci_src = sia if p == 0 else ria

                def chunk(k, _):
                    def adj(i, _):
                        sl = pl.ds(i * 16, 16)
                        src = pl.ds(k * C + i * 16, 16)
                        sg[sl] = sia[src] + c * N
                        rg[sl] = ria[src] + c * N
                        sci[sl] = sci_src[src]
                        return 0
                    lax.fori_loop(0, C // 16, adj, 0)
                    d1 = pltpu.async_copy(ps2_h.at[sg], psb, sem1)
                    d2 = pltpu.async_copy(pr2_h.at[rg], prb, sem2)
                    pltpu.sync_copy(
                        eq2_h.at[pl.ds(c * E + sbase + k * C, C)], eqb)
                    d1.wait()
                    d2.wait()

                    def comb(e, _):
                        for k2 in range(8):
                            sl = pl.ds(k2 * 16, 16)
                            v = eqb[e, sl] + psb[e, sl] + prb[e, sl]
                            eqb[e, sl] = jnp.maximum(v, 0.0)
                        return 0
                    lax.fori_loop(0, C, comb, 0)
                    pltpu.sync_copy(eqb, acc.at[sci], add=True)
                    return 0
                lax.fori_loop(0, SEG // C, chunk, 0)
                return 0
            lax.fori_loop(0, EPT // SEG, seg, 0)
            plsc.subcore_barrier()
            src = pl.ds(t * RPT, RPT)
            dst = pl.ds(c * NP + t * RPT, RPT)
            out_h = outs_h if p == 0 else outr_h
            pltpu.sync_copy(acc.at[src], out_h.at[dst])
            plsc.subcore_barrier()

    return pl.kernel(
        body,
        out_type=[jax.ShapeDtypeStruct((2 * NP, 128), jnp.float32)] * 2,
        mesh=mesh, scratch_types=scratch,
        compiler_params=pltpu.CompilerParams(needs_layout_passes=False))


def _sc_gnet(ps2, pr2, eq2, s_idx, r_idx):
    outs2, outr2 = _sc_gnet_call()(
        ps2.reshape(2 * N, 128), pr2.reshape(2 * N, 128),
        eq2.reshape(2 * E, 128), s_idx, r_idx)
    sa = outs2.reshape(2, NP, 128)[:, :N].transpose(1, 0, 2).reshape(N, 256)
    ra = outr2.reshape(2, NP, 128)[:, :N].transpose(1, 0, 2).reshape(N, 256)
    return sa, ra


def _sc_degrees_call():
    mesh = plsc.VectorSubcoreMesh(**_MESH)
    scratch = [
        pltpu.VMEM((NP,), jnp.float32),
        pltpu.VMEM((EPT,), jnp.int32),
        pltpu.VMEM((RPT,), jnp.float32),
        pltpu.VMEM((RPT,), jnp.float32),
        pltpu.VMEM_SHARED((NT, NP), jnp.float32),
    ]

    def body(s_h, r_h, sd_h, rd_h, part, iv, tmpv, comb, spm):
        c = lax.axis_index("c")
        t = lax.axis_index("s")

        def hist(idx_h, out_h):
            _zero_vmem(part, NP)
            ones16 = jnp.ones((16,), jnp.float32)
            pltpu.sync_copy(idx_h.at[pl.ds(t * EPT, EPT)], iv)

            def grp(i, _):
                plsc.addupdate_scatter(part, [iv[pl.ds(i * 16, 16)]],
                                       ones16)
                return 0
            lax.fori_loop(0, EPT // 16, grp, 0)
            pltpu.sync_copy(part, spm.at[t])
            plsc.subcore_barrier()
            _zero_vmem(comb, RPT)
            for p in range(NT):
                pltpu.sync_copy(spm.at[p, pl.ds(t * RPT, RPT)], tmpv)

                def acc(i, _):
                    sl = pl.ds(i * 16, 16)
                    comb[sl] = comb[sl] + tmpv[sl]
                    return 0
                lax.fori_loop(0, RPT // 16, acc, 0)
            pltpu.sync_copy(comb, out_h.at[pl.ds(t * RPT, RPT)])

        @pl.when(c == 0)
        def _():
            hist(s_h, sd_h)

        @pl.when(c == 1)
        def _():
            hist(r_h, rd_h)

    return pl.kernel(
        body,
        out_type=[jax.ShapeDtypeStruct((NP,), jnp.float32)] * 2,
        mesh=mesh, scratch_types=scratch,
        compiler_params=pltpu.CompilerParams(needs_layout_passes=False))


def _mm_body(nx, act, *refs):
    xs = refs[:nx]
    ws = refs[nx:2 * nx]
    b = refs[2 * nx]
    o = refs[2 * nx + 1]
    acc = jnp.zeros(o.shape, jnp.float32)
    for x, w in zip(xs, ws):
        acc = acc + jax.lax.dot_general(
            x[...], w[...], (((1,), (0,)), ((), ())),
            precision=HI, preferred_element_type=jnp.float32)
    acc = acc + b[...]
    if act == 'relu':
        acc = jnp.maximum(acc, 0.0)
    elif act == 'leaky':
        acc = jnp.where(acc >= 0, acc, 0.01 * acc)
    o[...] = acc


def _mm(xs_ws, b, act=None, bm=2000):
    xs = [x for x, _ in xs_ws]
    ws = [w for _, w in xs_ws]
    m = xs[0].shape[0]
    nout = ws[0].shape[1]
    grid = (m // bm,)
    in_specs = (
        [pl.BlockSpec((bm, x.shape[1]), lambda i: (i, 0)) for x in xs]
        + [pl.BlockSpec(w.shape, lambda i: (0, 0)) for w in ws]
        + [pl.BlockSpec((1, nout), lambda i: (0, 0))]
    )
    return pl.pallas_call(
        functools.partial(_mm_body, len(xs), act),
        grid=grid,
        in_specs=in_specs,
        out_specs=pl.BlockSpec((bm, nout), lambda i: (i, 0)),
        out_shape=jax.ShapeDtypeStruct((m, nout), jnp.float32),
    )(*xs, *ws, b.reshape(1, -1))


def _seg_sum(vals, idx, n):
    return jax.ops.segment_sum(vals, idx, num_segments=n)


def _seg_max(vals, idx, n):
    return jax.ops.segment_max(vals, idx, num_segments=n)


def _spmm(x, s_idx, r_idx, w=None):
    rows = x[s_idx]
    if w is not None:
        rows = rows * w[:, None]
    return _seg_sum(rows, r_idx, x.shape[0])


def _gat_weights(a_s, a_r, base, s_idx, r_idx):
    logits = a_s[s_idx] + a_r[r_idx] + base
    mx = _seg_max(logits, r_idx, N)
    e = jnp.exp(logits - mx[r_idx])
    den = _seg_sum(e, r_idx, N)
    return e / den[r_idx]


def kernel(nodes, edges, senders, receivers, globals_,
           W_gcn1, b_gcn1, W_gcn2, b_gcn2,
           Wq1, bq1, Wl1, bl1,
           Wq2, bq2, Wl2, bl2, Wn2, bn2,
           We, be, Wn, bn, Wg, bg):
    s_idx = senders.astype(jnp.int32)
    r_idx = receivers.astype(jnp.int32)
    sdeg, rdeg = _sc_degrees_call()(s_idx, r_idx)
    ssc = jax.lax.rsqrt(sdeg[:N, None] + 1.0)
    rsc = jax.lax.rsqrt(rdeg[:N, None] + 1.0)

    h = _mm([(nodes, W_gcn1)], b_gcn1, act='relu') * ssc
    h = (h + _sc_spmm(h, s_idx, r_idx)) * rsc
    h = _mm([(h, W_gcn2)], b_gcn2, act='relu') * ssc
    h = (h + _sc_spmm(h, s_idx, r_idx)) * rsc

    q = _mm([(h, Wq1)], bq1)
    wl_s, wl_r, wl_e = Wl1[:256, 0], Wl1[256:512, 0], Wl1[512:, 0]
    a_s = q @ wl_s + bl1[0]
    a_r = q @ wl_r
    w1 = _sc_gatw(a_s, a_r, _edge_base(edges, wl_e), s_idx, r_idx)
    h = _sc_spmm(q, s_idx, r_idx, w1)
    h = jnp.where(h >= 0, h, 0.01 * h)

    q = _mm([(h, Wq2)], bq2)
    wl_s, wl_r, wl_e = Wl2[:256, 0], Wl2[256:512, 0], Wl2[512:, 0]
    a_s = q @ wl_s + bl2[0]
    a_r = q @ wl_r
    w2 = _sc_gatw(a_s, a_r, _edge_base(edges, wl_e), s_idx, r_idx)
    agg = _sc_spmm(q, s_idx, r_idx, w2)
    h = _mm([(agg, Wn2)], bn2)

    We_e, We_s, We_r = We[:16], We[16:272], We[272:528]
    z256 = jnp.zeros((256,), jnp.float32)
    ps2 = _mm2(h, We_s, z256, bm=2000)
    pr2 = _mm2(h, We_r, z256, bm=2000)
    eq2 = _mm2(edges, We_e, be, bm=4000)
    sent_agg, recv_agg = _sc_gnet(ps2, pr2, eq2, s_idx, r_idx)

    Wn_h, Wn_s, Wn_r = Wn[:256], Wn[256:512], Wn[512:768]
    new_nodes = _mm([(h, Wn_h), (sent_agg, Wn_s), (recv_agg, Wn_r)], bn,
                    act='relu')

    node_attr = jnp.sum(new_nodes, axis=0, keepdims=True)
    edge_attr = jnp.sum(sent_agg, axis=0, keepdims=True)
    g = jnp.zeros((1, 64), jnp.float32)
    feats = jnp.concatenate([node_attr, edge_attr, g], axis=1)
    logits = jnp.maximum(feats @ Wg + bg, 0.0)
    return logits


# speedup vs baseline: 8.5698x; 1.1316x over previous
import functools
import jax
import jax.numpy as jnp
from jax import lax
from jax.experimental import pallas as pl
from jax.experimental.pallas import tpu as pltpu
from jax.experimental.pallas import tpu_sc as plsc

N = 10000
E = 320000
NP = 10240
NT = 16
C = 80
EPT = E // NT
SEG = 4000
RPT = NP // NT
HI = jax.lax.Precision.HIGHEST
_MESH = dict(core_axis_name="c", subcore_axis_name="s",
             num_cores=2, num_subcores=NT)


def _zero_vmem(ref, n):
    z = jnp.zeros((16,), jnp.float32)

    def zr(i, _):
        ref[pl.ds(i * 16, 16)] = z
        return 0
    lax.fori_loop(0, n // 16, zr, 0)


def _sc_spmm_call(weighted):
    mesh = plsc.VectorSubcoreMesh(**_MESH)
    scratch = [
        pltpu.VMEM((SEG,), jnp.int32),
        pltpu.VMEM((SEG,), jnp.int32),
        pltpu.VMEM((SEG,), jnp.float32) if weighted else None,
        pltpu.VMEM((C,), jnp.int32),
        pltpu.VMEM((C,), jnp.int32),
        pltpu.VMEM((C,), jnp.int32),
        pltpu.VMEM((C,), jnp.int32),
        pltpu.VMEM((C, 128), jnp.float32),
        pltpu.VMEM((C, 128), jnp.float32),
        pltpu.VMEM_SHARED((NP, 128), jnp.float32),
        pltpu.SemaphoreType.DMA,
        pltpu.SemaphoreType.DMA,
    ]
    scratch = [s for s in scratch if s is not None]

    def body(x2_h, s_h, r_h, w_h, ya_h, yb_h, *rest):
        if weighted:
            sia, ria, wa, sg0, sg1, rc0, rc1, rows0, rows1, acc, s0, s1 = rest
        else:
            sia, ria, sg0, sg1, rc0, rc1, rows0, rows1, acc, s0, s1 = rest
        c = lax.axis_index("c")
        t = lax.axis_index("s")

        def zrow(e, _):
            for k in range(8):
                rows0[e, pl.ds(k * 16, 16)] = jnp.zeros((16,), jnp.float32)
            return 0
        lax.fori_loop(0, C, zrow, 0)
        for j in range(RPT // C):
            pltpu.sync_copy(rows0, acc.at[pl.ds(t * RPT + j * C, C)])
        plsc.subcore_barrier()

        def seg(g, _):
            sbase = t * EPT + g * SEG
            pltpu.sync_copy(s_h.at[pl.ds(sbase, SEG)], sia)
            pltpu.sync_copy(r_h.at[pl.ds(sbase, SEG)], ria)
            if weighted:
                pltpu.sync_copy(w_h.at[pl.ds(sbase, SEG)], wa)

            def half_chunk(k, sg, rc, rows, sem):
                def adj(i, _):
                    sl = pl.ds(i * 16, 16)
                    src = pl.ds(k * C + i * 16, 16)
                    sg[sl] = sia[src] * 2 + c
                    rc[sl] = ria[src]
                    return 0
                lax.fori_loop(0, C // 16, adj, 0)
                return pltpu.async_copy(x2_h.at[sg], rows, sem)

            def drain(k, rc, rows):
                if weighted:
                    def scale(i, _):
                        wvec = wa[pl.ds(k * C + i * 16, 16)]
                        for j in range(16):
                            we = wvec[j]
                            for k2 in range(8):
                                sl = pl.ds(k2 * 16, 16)
                                rows[i * 16 + j, sl] = (
                                    rows[i * 16 + j, sl] * we)
                        return 0
                    lax.fori_loop(0, C // 16, scale, 0)
                pltpu.sync_copy(rows, acc.at[rc], add=True)

            def pair(p, _):
                k0 = 2 * p
                d0 = half_chunk(k0, sg0, rc0, rows0, s0)
                d1 = half_chunk(k0 + 1, sg1, rc1, rows1, s1)
                d0.wait()
                drain(k0, rc0, rows0)
                d1.wait()
                drain(k0 + 1, rc1, rows1)
                return 0
            lax.fori_loop(0, SEG // C // 2, pair, 0)
            return 0
        lax.fori_loop(0, EPT // SEG, seg, 0)
        plsc.subcore_barrier()
        sl = pl.ds(t * RPT, RPT)

        @pl.when(c == 0)
        def _():
            pltpu.sync_copy(acc.at[sl], ya_h.at[sl])

        @pl.when(c == 1)
        def _():
            pltpu.sync_copy(acc.at[sl], yb_h.at[sl])

    return pl.kernel(
        body,
        out_type=[jax.ShapeDtypeStruct((NP, 128), jnp.float32)] * 2,
        mesh=mesh, scratch_types=scratch,
        compiler_params=pltpu.CompilerParams(needs_layout_passes=False))


def _sc_spmm(x, s_idx, r_idx, w=None):
    x2 = x.reshape(2 * N, 128)
    if w is None:
        w = jnp.zeros((8,), jnp.float32)
    ya, yb = _sc_spmm_call(w.shape[0] == E)(x2, s_idx, r_idx, w)
    return jnp.concatenate([ya[:N], yb[:N]], axis=1)


def _sc_gatw_call():
    mesh = plsc.VectorSubcoreMesh(**_MESH)
    HALF = EPT // 2
    scratch = [
        pltpu.VMEM((NP,), jnp.float32),
        pltpu.VMEM((NP,), jnp.float32),
        pltpu.VMEM((NP,), jnp.float32),
        pltpu.VMEM((EPT,), jnp.float32),
        pltpu.VMEM((EPT,), jnp.float32),
        pltpu.VMEM((EPT,), jnp.int32),
        pltpu.VMEM((EPT,), jnp.int32),
        pltpu.VMEM((C,), jnp.float32),
        pltpu.VMEM((RPT,), jnp.float32),
        pltpu.VMEM((RPT,), jnp.float32),
        pltpu.VMEM_SHARED((NT, NP), jnp.float32),
        pltpu.VMEM_SHARED((NP,), jnp.float32),
    ]

    def body(as_h, ar_h, b_h, s_h, r_h, w_h,
             asv, arv, den, exv, bba, sia, ria, wbuf, tmpv, comb,
             spm, spm_den):
        c = lax.axis_index("c")
        t = lax.axis_index("s")
        pltpu.sync_copy(as_h, asv)
        pltpu.sync_copy(ar_h, arv)
        pltpu.sync_copy(s_h.at[pl.ds(t * EPT, EPT)], sia)
        pltpu.sync_copy(r_h.at[pl.ds(t * EPT, EPT)], ria)
        pltpu.sync_copy(b_h.at[pl.ds(t * EPT, EPT)], bba)
        _zero_vmem(den, NP)

        def grp(i, _):
            sl = pl.ds(i * 16, 16)
            svec = sia[sl]
            rvec = ria[sl]
            sa = plsc.load_gather(asv, [svec])
            sr = plsc.load_gather(arv, [rvec])
            ex16 = jnp.exp(sa + sr + bba[sl])
            exv[sl] = ex16
            plsc.addupdate_scatter(den, [rvec], ex16)
            return 0
        lax.fori_loop(0, EPT // 16, grp, 0)

        pltpu.sync_copy(den, spm.at[t])
        plsc.subcore_barrier()
        _zero_vmem(comb, RPT)
        for p in range(NT):
            pltpu.sync_copy(spm.at[p, pl.ds(t * RPT, RPT)], tmpv)

            def acc(i, _):
                sl = pl.ds(i * 16, 16)
                comb[sl] = comb[sl] + tmpv[sl]
                return 0
            lax.fori_loop(0, RPT // 16, acc, 0)
        pltpu.sync_copy(comb, spm_den.at[pl.ds(t * RPT, RPT)])
        plsc.subcore_barrier()
        pltpu.sync_copy(spm_den, den)

        def chunk2(k, _):
            off = c * HALF + k * C

            def grp2(i, _):
                sl = pl.ds(i * 16, 16)
                src = pl.ds(off + i * 16, 16)
                dv = plsc.load_gather(den, [ria[src]])
                wbuf[sl] = exv[src] / dv
                return 0
            lax.fori_loop(0, C // 16, grp2, 0)
            pltpu.sync_copy(wbuf, w_h.at[pl.ds(t * EPT + off, C)])
            return 0
        lax.fori_loop(0, HALF // C, chunk2, 0)

    return pl.kernel(
        body,
        out_type=jax.ShapeDtypeStruct((E,), jnp.float32),
        mesh=mesh, scratch_types=scratch,
        compiler_params=pltpu.CompilerParams(needs_layout_passes=False))


def _sc_gatw(a_s, a_r, base, s_idx, r_idx):
    asp = jnp.pad(a_s, (0, NP - N))
    arp = jnp.pad(a_r, (0, NP - N))
    return _sc_gatw_call()(asp, arp, base, s_idx, r_idx)


def _edge_base(edges, wl_e):
    wpad = jnp.zeros((16, 128), jnp.float32).at[:, 0].set(wl_e)
    out = _mm([(edges, wpad)], jnp.zeros((128,), jnp.float32), bm=4000)
    return out[:, 0]


def _mm2_body(x_ref, w_ref, b_ref, o_ref):
    acc = jax.lax.dot_general(
        x_ref[...], w_ref[0], (((1,), (0,)), ((), ())),
        precision=HI, preferred_element_type=jnp.float32)
    o_ref[0] = acc + b_ref[0]


def _mm2(x, w, b, bm):
    m = x.shape[0]
    k = x.shape[1]
    w2 = w.reshape(k, 2, 128).transpose(1, 0, 2)
    b2 = b.reshape(1, 2, 128).transpose(1, 0, 2)
    return pl.pallas_call(
        _mm2_body,
        grid=(m // bm, 2),
        in_specs=[pl.BlockSpec((bm, k), lambda i, q: (i, 0)),
                  pl.BlockSpec((1, k, 128), lambda i, q: (q, 0, 0)),
                  pl.BlockSpec((1, 1, 128), lambda i, q: (q, 0, 0))],
        out_specs=pl.BlockSpec((1, bm, 128), lambda i, q: (q, i, 0)),
        out_shape=jax.ShapeDtypeStruct((2, m, 128), jnp.float32),
    )(x, w2, b2)


def _sc_gnet_call():
    mesh = plsc.VectorSubcoreMesh(**_MESH)
    scratch = [
        pltpu.VMEM((SEG,), jnp.int32),
        pltpu.VMEM((SEG,), jnp.int32),
        pltpu.VMEM((C,), jnp.int32),
        pltpu.VMEM((C,), jnp.int32),
        pltpu.VMEM((C,), jnp.int32),
        pltpu.VMEM((C, 128), jnp.float32),
        pltpu.VMEM((C, 128), jnp.float32),
        pltpu.VMEM((C, 128), jnp.float32),
        pltpu.VMEM_SHARED((NP, 128), jnp.float32),
        pltpu.SemaphoreType.DMA,
        pltpu.SemaphoreType.DMA,
    ]

    def body(ps2_h, pr2_h, eq2_h, s_h, r_h, outs_h, outr_h, t2_h,
             sia, ria, sg, rg, sci, eqb, psb, prb, acc, sem1, sem2):
        c = lax.axis_index("c")
        t = lax.axis_index("s")
        for p in range(2):
            def zrow(e, _):
                for k in range(8):
                    eqb[e, pl.ds(k * 16, 16)] = jnp.zeros((16,), jnp.float32)
                return 0
            lax.fori_loop(0, C, zrow, 0)
            for jz in range(RPT // C):
                pltpu.sync_copy(eqb, acc.at[pl.ds(t * RPT + jz * C, C)])
            plsc.subcore_barrier()

            def seg(g, _):
                sbase = t * EPT + g * SEG
                if p == 0:
                    pltpu.sync_copy(s_h.at[pl.ds(sbase, SEG)], sia)
                    pltpu.sync_copy(r_h.at[pl.ds(sbase, SEG)], ria)
                else:
                    pltpu.sync_copy(r_h.at[pl.ds(sbase, SEG)], ria)

                def chunk(k, _):
                    lin = pl.ds(c * E + sbase + k * C, C)
                    if p == 0:
                        def adj(i, _):
                            sl = pl.ds(i * 16, 16)
                            src = pl.ds(k * C + i * 16, 16)
                            sg[sl] = sia[src] + c * N
                            rg[sl] = ria[src] + c * N
                            sci[sl] = sia[src]
                            return 0
                        lax.fori_loop(0, C // 16, adj, 0)
                        d1 = pltpu.async_copy(ps2_h.at[sg], psb, sem1)
                        d2 = pltpu.async_copy(pr2_h.at[rg], prb, sem2)
                        pltpu.sync_copy(eq2_h.at[lin], eqb)
                        d1.wait()
                        d2.wait()

                        def comb(e, _):
                            for k2 in range(8):
                                sl = pl.ds(k2 * 16, 16)
                                v = eqb[e, sl] + psb[e, sl] + prb[e, sl]
                                eqb[e, sl] = jnp.maximum(v, 0.0)
                            return 0
                        lax.fori_loop(0, C, comb, 0)
                        d3 = pltpu.async_copy(eqb, t2_h.at[lin], sem1)
                        pltpu.sync_copy(eqb, acc.at[sci], add=True)
                        d3.wait()
                    else:
                        def adj(i, _):
                            sl = pl.ds(i * 16, 16)
                            src = pl.ds(k * C + i * 16, 16)
                            sci[sl] = ria[src]
                            return 0
                        lax.fori_loop(0, C // 16, adj, 0)
                        pltpu.sync_copy(t2_h.at[lin], eqb)
                        pltpu.sync_copy(eqb, acc.at[sci], add=True)
                    return 0
                lax.fori_loop(0, SEG // C, chunk, 0)
                return 0
            lax.fori_loop(0, EPT // SEG, seg, 0)
            plsc.subcore_barrier()
            src = pl.ds(t * RPT, RPT)
            dst = pl.ds(c * NP + t * RPT, RPT)
            out_h = outs_h if p == 0 else outr_h
            pltpu.sync_copy(acc.at[src], out_h.at[dst])
            plsc.subcore_barrier()

    return pl.kernel(
        body,
        out_type=[jax.ShapeDtypeStruct((2 * NP, 128), jnp.float32)] * 2
        + [jax.ShapeDtypeStruct((2 * E, 128), jnp.float32)],
        mesh=mesh, scratch_types=scratch,
        compiler_params=pltpu.CompilerParams(needs_layout_passes=False))


def _sc_gnet(ps2, pr2, eq2, s_idx, r_idx):
    outs2, outr2, _ = _sc_gnet_call()(
        ps2.reshape(2 * N, 128), pr2.reshape(2 * N, 128),
        eq2.reshape(2 * E, 128), s_idx, r_idx)
    sa = outs2.reshape(2, NP, 128)[:, :N].transpose(1, 0, 2).reshape(N, 256)
    ra = outr2.reshape(2, NP, 128)[:, :N].transpose(1, 0, 2).reshape(N, 256)
    return sa, ra


def _sc_degrees_call():
    mesh = plsc.VectorSubcoreMesh(**_MESH)
    scratch = [
        pltpu.VMEM((NP,), jnp.float32),
        pltpu.VMEM((EPT,), jnp.int32),
        pltpu.VMEM((RPT,), jnp.float32),
        pltpu.VMEM((RPT,), jnp.float32),
        pltpu.VMEM_SHARED((NT, NP), jnp.float32),
    ]

    def body(s_h, r_h, sd_h, rd_h, part, iv, tmpv, comb, spm):
        c = lax.axis_index("c")
        t = lax.axis_index("s")

        def hist(idx_h, out_h):
            _zero_vmem(part, NP)
            ones16 = jnp.ones((16,), jnp.float32)
            pltpu.sync_copy(idx_h.at[pl.ds(t * EPT, EPT)], iv)

            def grp(i, _):
                plsc.addupdate_scatter(part, [iv[pl.ds(i * 16, 16)]],
                                       ones16)
                return 0
            lax.fori_loop(0, EPT // 16, grp, 0)
            pltpu.sync_copy(part, spm.at[t])
            plsc.subcore_barrier()
            _zero_vmem(comb, RPT)
            for p in range(NT):
                pltpu.sync_copy(spm.at[p, pl.ds(t * RPT, RPT)], tmpv)

                def acc(i, _):
                    sl = pl.ds(i * 16, 16)
                    comb[sl] = comb[sl] + tmpv[sl]
                    return 0
                lax.fori_loop(0, RPT // 16, acc, 0)
            pltpu.sync_copy(comb, out_h.at[pl.ds(t * RPT, RPT)])

        @pl.when(c == 0)
        def _():
            hist(s_h, sd_h)

        @pl.when(c == 1)
        def _():
            hist(r_h, rd_h)

    return pl.kernel(
        body,
        out_type=[jax.ShapeDtypeStruct((NP,), jnp.float32)] * 2,
        mesh=mesh, scratch_types=scratch,
        compiler_params=pltpu.CompilerParams(needs_layout_passes=False))


def _mm_body(nx, act, *refs):
    xs = refs[:nx]
    ws = refs[nx:2 * nx]
    b = refs[2 * nx]
    o = refs[2 * nx + 1]
    acc = jnp.zeros(o.shape, jnp.float32)
    for x, w in zip(xs, ws):
        acc = acc + jax.lax.dot_general(
            x[...], w[...], (((1,), (0,)), ((), ())),
            precision=HI, preferred_element_type=jnp.float32)
    acc = acc + b[...]
    if act == 'relu':
        acc = jnp.maximum(acc, 0.0)
    elif act == 'leaky':
        acc = jnp.where(acc >= 0, acc, 0.01 * acc)
    o[...] = acc


def _mm(xs_ws, b, act=None, bm=2000):
    xs = [x for x, _ in xs_ws]
    ws = [w for _, w in xs_ws]
    m = xs[0].shape[0]
    nout = ws[0].shape[1]
    grid = (m // bm,)
    in_specs = (
        [pl.BlockSpec((bm, x.shape[1]), lambda i: (i, 0)) for x in xs]
        + [pl.BlockSpec(w.shape, lambda i: (0, 0)) for w in ws]
        + [pl.BlockSpec((1, nout), lambda i: (0, 0))]
    )
    return pl.pallas_call(
        functools.partial(_mm_body, len(xs), act),
        grid=grid,
        in_specs=in_specs,
        out_specs=pl.BlockSpec((bm, nout), lambda i: (i, 0)),
        out_shape=jax.ShapeDtypeStruct((m, nout), jnp.float32),
    )(*xs, *ws, b.reshape(1, -1))


def _seg_sum(vals, idx, n):
    return jax.ops.segment_sum(vals, idx, num_segments=n)


def _seg_max(vals, idx, n):
    return jax.ops.segment_max(vals, idx, num_segments=n)


def _spmm(x, s_idx, r_idx, w=None):
    rows = x[s_idx]
    if w is not None:
        rows = rows * w[:, None]
    return _seg_sum(rows, r_idx, x.shape[0])


def _gat_weights(a_s, a_r, base, s_idx, r_idx):
    logits = a_s[s_idx] + a_r[r_idx] + base
    mx = _seg_max(logits, r_idx, N)
    e = jnp.exp(logits - mx[r_idx])
    den = _seg_sum(e, r_idx, N)
    return e / den[r_idx]


def kernel(nodes, edges, senders, receivers, globals_,
           W_gcn1, b_gcn1, W_gcn2, b_gcn2,
           Wq1, bq1, Wl1, bl1,
           Wq2, bq2, Wl2, bl2, Wn2, bn2,
           We, be, Wn, bn, Wg, bg):
    s_idx = senders.astype(jnp.int32)
    r_idx = receivers.astype(jnp.int32)
    sdeg, rdeg = _sc_degrees_call()(s_idx, r_idx)
    ssc = jax.lax.rsqrt(sdeg[:N, None] + 1.0)
    rsc = jax.lax.rsqrt(rdeg[:N, None] + 1.0)

    h = _mm([(nodes, W_gcn1)], b_gcn1, act='relu') * ssc
    h = (h + _sc_spmm(h, s_idx, r_idx)) * rsc
    h = _mm([(h, W_gcn2)], b_gcn2, act='relu') * ssc
    h = (h + _sc_spmm(h, s_idx, r_idx)) * rsc

    q = _mm([(h, Wq1)], bq1)
    wl_s, wl_r, wl_e = Wl1[:256, 0], Wl1[256:512, 0], Wl1[512:, 0]
    a_s = q @ wl_s + bl1[0]
    a_r = q @ wl_r
    w1 = _sc_gatw(a_s, a_r, _edge_base(edges, wl_e), s_idx, r_idx)
    h = _sc_spmm(q, s_idx, r_idx, w1)
    h = jnp.where(h >= 0, h, 0.01 * h)

    q = _mm([(h, Wq2)], bq2)
    wl_s, wl_r, wl_e = Wl2[:256, 0], Wl2[256:512, 0], Wl2[512:, 0]
    a_s = q @ wl_s + bl2[0]
    a_r = q @ wl_r
    w2 = _sc_gatw(a_s, a_r, _edge_base(edges, wl_e), s_idx, r_idx)
    agg = _sc_spmm(q, s_idx, r_idx, w2)
    h = _mm([(agg, Wn2)], bn2)

    We_e, We_s, We_r = We[:16], We[16:272], We[272:528]
    z256 = jnp.zeros((256,), jnp.float32)
    ps2 = _mm2(h, We_s, z256, bm=2000)
    pr2 = _mm2(h, We_r, z256, bm=2000)
    eq2 = _mm2(edges, We_e, be, bm=4000)
    sent_agg, recv_agg = _sc_gnet(ps2, pr2, eq2, s_idx, r_idx)

    Wn_h, Wn_s, Wn_r = Wn[:256], Wn[256:512], Wn[512:768]
    new_nodes = _mm([(h, Wn_h), (sent_agg, Wn_s), (recv_agg, Wn_r)], bn,
                    act='relu')

    node_attr = jnp.sum(new_nodes, axis=0, keepdims=True)
    edge_attr = jnp.sum(sent_agg, axis=0, keepdims=True)
    g = jnp.zeros((1, 64), jnp.float32)
    feats = jnp.concatenate([node_attr, edge_attr, g], axis=1)
    logits = jnp.maximum(feats @ Wg + bg, 0.0)
    return logits


# speedup vs baseline: 8.5773x; 1.0009x over previous
import functools
import jax
import jax.numpy as jnp
from jax import lax
from jax.experimental import pallas as pl
from jax.experimental.pallas import tpu as pltpu
from jax.experimental.pallas import tpu_sc as plsc

N = 10000
E = 320000
NP = 10240
NT = 16
C = 80
EPT = E // NT
SEG = 4000
RPT = NP // NT
HI = jax.lax.Precision.HIGHEST
_MESH = dict(core_axis_name="c", subcore_axis_name="s",
             num_cores=2, num_subcores=NT)


def _zero_vmem(ref, n):
    z = jnp.zeros((16,), jnp.float32)

    def zr(i, _):
        ref[pl.ds(i * 16, 16)] = z
        return 0
    lax.fori_loop(0, n // 16, zr, 0)


def _sc_spmm_call(weighted):
    mesh = plsc.VectorSubcoreMesh(**_MESH)
    scratch = [
        pltpu.VMEM((SEG,), jnp.int32),
        pltpu.VMEM((SEG,), jnp.int32),
        pltpu.VMEM((SEG,), jnp.float32) if weighted else None,
        pltpu.VMEM((C,), jnp.int32),
        pltpu.VMEM((C,), jnp.int32),
        pltpu.VMEM((C,), jnp.int32),
        pltpu.VMEM((C,), jnp.int32),
        pltpu.VMEM((C, 128), jnp.float32),
        pltpu.VMEM((C, 128), jnp.float32),
        pltpu.VMEM_SHARED((NP, 128), jnp.float32),
        pltpu.SemaphoreType.DMA,
        pltpu.SemaphoreType.DMA,
    ]
    scratch = [s for s in scratch if s is not None]

    def body(x2_h, s_h, r_h, w_h, ya_h, yb_h, *rest):
        if weighted:
            sia, ria, wa, sg0, sg1, rc0, rc1, rows0, rows1, acc, s0, s1 = rest
        else:
            sia, ria, sg0, sg1, rc0, rc1, rows0, rows1, acc, s0, s1 = rest
        c = lax.axis_index("c")
        t = lax.axis_index("s")

        def zrow(e, _):
            for k in range(8):
                rows0[e, pl.ds(k * 16, 16)] = jnp.zeros((16,), jnp.float32)
            return 0
        lax.fori_loop(0, C, zrow, 0)
        for j in range(RPT // C):
            pltpu.sync_copy(rows0, acc.at[pl.ds(t * RPT + j * C, C)])
        plsc.subcore_barrier()

        def seg(g, _):
            sbase = t * EPT + g * SEG
            pltpu.sync_copy(s_h.at[pl.ds(sbase, SEG)], sia)
            pltpu.sync_copy(r_h.at[pl.ds(sbase, SEG)], ria)
            if weighted:
                pltpu.sync_copy(w_h.at[pl.ds(sbase, SEG)], wa)

            def half_chunk(k, sg, rc, rows, sem):
                def adj(i, _):
                    sl = pl.ds(i * 16, 16)
                    src = pl.ds(k * C + i * 16, 16)
                    sg[sl] = sia[src] * 2 + c
                    rc[sl] = ria[src]
                    return 0
                lax.fori_loop(0, C // 16, adj, 0)
                return pltpu.async_copy(x2_h.at[sg], rows, sem)

            def drain(k, rc, rows):
                if weighted:
                    def scale(i, _):
                        wvec = wa[pl.ds(k * C + i * 16, 16)]
                        for j in range(16):
                            we = wvec[j]
                            for k2 in range(8):
                                sl = pl.ds(k2 * 16, 16)
                                rows[i * 16 + j, sl] = (
                                    rows[i * 16 + j, sl] * we)
                        return 0
                    lax.fori_loop(0, C // 16, scale, 0)
                pltpu.sync_copy(rows, acc.at[rc], add=True)

            def pair(p, _):
                k0 = 2 * p
                d0 = half_chunk(k0, sg0, rc0, rows0, s0)
                d1 = half_chunk(k0 + 1, sg1, rc1, rows1, s1)
                d0.wait()
                drain(k0, rc0, rows0)
                d1.wait()
                drain(k0 + 1, rc1, rows1)
                return 0
            lax.fori_loop(0, SEG // C // 2, pair, 0)
            return 0
        lax.fori_loop(0, EPT // SEG, seg, 0)
        plsc.subcore_barrier()
        sl = pl.ds(t * RPT, RPT)

        @pl.when(c == 0)
        def _():
            pltpu.sync_copy(acc.at[sl], ya_h.at[sl])

        @pl.when(c == 1)
        def _():
            pltpu.sync_copy(acc.at[sl], yb_h.at[sl])

    return pl.kernel(
        body,
        out_type=[jax.ShapeDtypeStruct((NP, 128), jnp.float32)] * 2,
        mesh=mesh, scratch_types=scratch,
        compiler_params=pltpu.CompilerParams(needs_layout_passes=False))


def _sc_spmm(x, s_idx, r_idx, w=None):
    x2 = x.reshape(2 * N, 128)
    if w is None:
        w = jnp.zeros((8,), jnp.float32)
    ya, yb = _sc_spmm_call(w.shape[0] == E)(x2, s_idx, r_idx, w)
    return jnp.concatenate([ya[:N], yb[:N]], axis=1)


def _sc_gatw_call():
    mesh = plsc.VectorSubcoreMesh(**_MESH)
    HALF = EPT // 2
    scratch = [
        pltpu.VMEM((NP,), jnp.float32),
        pltpu.VMEM((NP,), jnp.float32),
        pltpu.VMEM((NP,), jnp.float32),
        pltpu.VMEM((EPT,), jnp.float32),
        pltpu.VMEM((EPT,), jnp.float32),
        pltpu.VMEM((EPT,), jnp.int32),
        pltpu.VMEM((EPT,), jnp.int32),
        pltpu.VMEM((C,), jnp.float32),
        pltpu.VMEM((RPT,), jnp.float32),
        pltpu.VMEM((RPT,), jnp.float32),
        pltpu.VMEM_SHARED((NT, NP), jnp.float32),
        pltpu.VMEM_SHARED((NP,), jnp.float32),
    ]

    def body(as_h, ar_h, b_h, s_h, r_h, w_h,
             asv, arv, den, exv, bba, sia, ria, wbuf, tmpv, comb,
             spm, spm_den):
        c = lax.axis_index("c")
        t = lax.axis_index("s")
        pltpu.sync_copy(as_h, asv)
        pltpu.sync_copy(ar_h, arv)
        pltpu.sync_copy(s_h.at[pl.ds(t * EPT, EPT)], sia)
        pltpu.sync_copy(r_h.at[pl.ds(t * EPT, EPT)], ria)
        pltpu.sync_copy(b_h.at[pl.ds(t * EPT, EPT)], bba)
        _zero_vmem(den, NP)

        def grp(i, _):
            sl = pl.ds(i * 16, 16)
            svec = sia[sl]
            rvec = ria[sl]
            sa = plsc.load_gather(asv, [svec])
            sr = plsc.load_gather(arv, [rvec])
            ex16 = jnp.exp(sa + sr + bba[sl])
            exv[sl] = ex16
            plsc.addupdate_scatter(den, [rvec], ex16)
            return 0
        lax.fori_loop(0, EPT // 16, grp, 0)

        pltpu.sync_copy(den, spm.at[t])
        plsc.subcore_barrier()
        _zero_vmem(comb, RPT)
        for p in range(NT):
            pltpu.sync_copy(spm.at[p, pl.ds(t * RPT, RPT)], tmpv)

            def acc(i, _):
                sl = pl.ds(i * 16, 16)
                comb[sl] = comb[sl] + tmpv[sl]
                return 0
            lax.fori_loop(0, RPT // 16, acc, 0)
        pltpu.sync_copy(comb, spm_den.at[pl.ds(t * RPT, RPT)])
        plsc.subcore_barrier()
        pltpu.sync_copy(spm_den, den)

        def chunk2(k, _):
            off = c * HALF + k * C

            def grp2(i, _):
                sl = pl.ds(i * 16, 16)
                src = pl.ds(off + i * 16, 16)
                dv = plsc.load_gather(den, [ria[src]])
                wbuf[sl] = exv[src] / dv
                return 0
            lax.fori_loop(0, C // 16, grp2, 0)
            pltpu.sync_copy(wbuf, w_h.at[pl.ds(t * EPT + off, C)])
            return 0
        lax.fori_loop(0, HALF // C, chunk2, 0)

    return pl.kernel(
        body,
        out_type=jax.ShapeDtypeStruct((E,), jnp.float32),
        mesh=mesh, scratch_types=scratch,
        compiler_params=pltpu.CompilerParams(needs_layout_passes=False))


def _sc_gatw(a_s, a_r, base, s_idx, r_idx):
    asp = jnp.pad(a_s, (0, NP - N))
    arp = jnp.pad(a_r, (0, NP - N))
    return _sc_gatw_call()(asp, arp, base, s_idx, r_idx)


def _edge_base(edges, wl_e):
    wpad = jnp.zeros((16, 128), jnp.float32).at[:, 0].set(wl_e)
    out = _mm([(edges, wpad)], jnp.zeros((128,), jnp.float32), bm=4000)
    return out[:, 0]


def _mm2_body(x_ref, w_ref, b_ref, o_ref):
    acc = jax.lax.dot_general(
        x_ref[...], w_ref[0], (((1,), (0,)), ((), ())),
        precision=HI, preferred_element_type=jnp.float32)
    o_ref[0] = acc + b_ref[0]


def _mm2(x, w, b, bm):
    m = x.shape[0]
    k = x.shape[1]
    w2 = w.reshape(k, 2, 128).transpose(1, 0, 2)
    b2 = b.reshape(1, 2, 128).transpose(1, 0, 2)
    return pl.pallas_call(
        _mm2_body,
        grid=(m // bm, 2),
        in_specs=[pl.BlockSpec((bm, k), lambda i, q: (i, 0)),
                  pl.BlockSpec((1, k, 128), lambda i, q: (q, 0, 0)),
                  pl.BlockSpec((1, 1, 128), lambda i, q: (q, 0, 0))],
        out_specs=pl.BlockSpec((1, bm, 128), lambda i, q: (q, i, 0)),
        out_shape=jax.ShapeDtypeStruct((2, m, 128), jnp.float32),
    )(x, w2, b2)


def _sc_gnet_call():
    mesh = plsc.VectorSubcoreMesh(**_MESH)
    scratch = [
        pltpu.VMEM((SEG,), jnp.int32),
        pltpu.VMEM((SEG,), jnp.int32),
        pltpu.VMEM((C,), jnp.int32),
        pltpu.VMEM((C,), jnp.int32),
        pltpu.VMEM((C,), jnp.int32),
        pltpu.VMEM((C, 128), jnp.float32),
        pltpu.VMEM((C, 128), jnp.float32),
        pltpu.VMEM((C, 128), jnp.float32),
        pltpu.VMEM_SHARED((NP, 128), jnp.float32),
        pltpu.SemaphoreType.DMA,
        pltpu.SemaphoreType.DMA,
    ]

    def body(ps2_h, pr2_h, eq2_h, s_h, r_h, outs_h, outr_h, t2_h,
             sia, ria, sg, rg, sci, eqb, psb, prb, acc, sem1, sem2):
        c = lax.axis_index("c")
        t = lax.axis_index("s")
        for p in range(2):
            def zrow(e, _):
                for k in range(8):
                    eqb[e, pl.ds(k * 16, 16)] = jnp.zeros((16,), jnp.float32)
                return 0
            lax.fori_loop(0, C, zrow, 0)
            for jz in range(RPT // C):
                pltpu.sync_copy(eqb, acc.at[pl.ds(t * RPT + jz * C, C)])
            plsc.subcore_barrier()

            def seg(g, _):
                sbase = t * EPT + g * SEG
                if p == 0:
                    pltpu.sync_copy(s_h.at[pl.ds(sbase, SEG)], sia)
                    pltpu.sync_copy(r_h.at[pl.ds(sbase, SEG)], ria)
                else:
                    pltpu.sync_copy(r_h.at[pl.ds(sbase, SEG)], ria)

                def chunk(k, _):
                    lin = pl.ds(c * E + sbase + k * C, C)
                    if p == 0:
                        def adj(i, _):
                            sl = pl.ds(i * 16, 16)
                            src = pl.ds(k * C + i * 16, 16)
                            sg[sl] = sia[src] + c * N
                            rg[sl] = ria[src] + c * N
                            sci[sl] = sia[src]
                            return 0
                        lax.fori_loop(0, C // 16, adj, 0)
                        d1 = pltpu.async_copy(ps2_h.at[sg], psb, sem1)
                        d2 = pltpu.async_copy(pr2_h.at[rg], prb, sem2)
                        pltpu.sync_copy(eq2_h.at[lin], eqb)
                        d1.wait()
                        d2.wait()

                        def comb(e, _):
                            for k2 in range(8):
                                sl = pl.ds(k2 * 16, 16)
                                v = eqb[e, sl] + psb[e, sl] + prb[e, sl]
                                eqb[e, sl] = jnp.maximum(v, 0.0)
                            return 0
                        lax.fori_loop(0, C, comb, 0)
                        d3 = pltpu.async_copy(eqb, t2_h.at[lin], sem1)
                        pltpu.sync_copy(eqb, acc.at[sci], add=True)
                        d3.wait()
                    else:
                        def adj(i, _):
                            sl = pl.ds(i * 16, 16)
                            src = pl.ds(k * C + i * 16, 16)
                            sci[sl] = ria[src]
                            return 0
                        lax.fori_loop(0, C // 16, adj, 0)
                        pltpu.sync_copy(t2_h.at[lin], eqb)
                        pltpu.sync_copy(eqb, acc.at[sci], add=True)
                    return 0
                lax.fori_loop(0, SEG // C, chunk, 0)
                return 0
            lax.fori_loop(0, EPT // SEG, seg, 0)
            plsc.subcore_barrier()
            src = pl.ds(t * RPT, RPT)
            dst = pl.ds(c * NP + t * RPT, RPT)
            out_h = outs_h if p == 0 else outr_h
            pltpu.sync_copy(acc.at[src], out_h.at[dst])
            plsc.subcore_barrier()

    return pl.kernel(
        body,
        out_type=[jax.ShapeDtypeStruct((2 * NP, 128), jnp.float32)] * 2
        + [jax.ShapeDtypeStruct((2 * E, 128), jnp.float32)],
        mesh=mesh, scratch_types=scratch,
        compiler_params=pltpu.CompilerParams(needs_layout_passes=False))


def _sc_gnet(ps2, pr2, eq2, s_idx, r_idx):
    outs2, outr2, _ = _sc_gnet_call()(
        ps2.reshape(2 * N, 128), pr2.reshape(2 * N, 128),
        eq2.reshape(2 * E, 128), s_idx, r_idx)
    sa = outs2.reshape(2, NP, 128)[:, :N].transpose(1, 0, 2).reshape(N, 256)
    ra = outr2.reshape(2, NP, 128)[:, :N].transpose(1, 0, 2).reshape(N, 256)
    return sa, ra


def _sc_degrees_call():
    mesh = plsc.VectorSubcoreMesh(**_MESH)
    scratch = [
        pltpu.VMEM((NP,), jnp.float32),
        pltpu.VMEM((EPT,), jnp.int32),
        pltpu.VMEM((RPT,), jnp.float32),
        pltpu.VMEM((RPT,), jnp.float32),
        pltpu.VMEM_SHARED((NT, NP), jnp.float32),
    ]

    def body(s_h, r_h, sd_h, rd_h, part, iv, tmpv, comb, spm):
        c = lax.axis_index("c")
        t = lax.axis_index("s")

        def hist(idx_h, out_h):
            _zero_vmem(part, NP)
            ones16 = jnp.ones((16,), jnp.float32)
            pltpu.sync_copy(idx_h.at[pl.ds(t * EPT, EPT)], iv)

            def grp(i, _):
                plsc.addupdate_scatter(part, [iv[pl.ds(i * 16, 16)]],
                                       ones16)
                return 0
            lax.fori_loop(0, EPT // 16, grp, 0)
            pltpu.sync_copy(part, spm.at[t])
            plsc.subcore_barrier()
            _zero_vmem(comb, RPT)
            for p in range(NT):
                pltpu.sync_copy(spm.at[p, pl.ds(t * RPT, RPT)], tmpv)

                def acc(i, _):
                    sl = pl.ds(i * 16, 16)
                    comb[sl] = comb[sl] + tmpv[sl]
                    return 0
                lax.fori_loop(0, RPT // 16, acc, 0)
            pltpu.sync_copy(comb, out_h.at[pl.ds(t * RPT, RPT)])

        @pl.when(c == 0)
        def _():
            hist(s_h, sd_h)

        @pl.when(c == 1)
        def _():
            hist(r_h, rd_h)

    return pl.kernel(
        body,
        out_type=[jax.ShapeDtypeStruct((NP,), jnp.float32)] * 2,
        mesh=mesh, scratch_types=scratch,
        compiler_params=pltpu.CompilerParams(needs_layout_passes=False))


def _mm_body(nx, act, *refs):
    xs = refs[:nx]
    ws = refs[nx:2 * nx]
    b = refs[2 * nx]
    o = refs[2 * nx + 1]
    acc = jnp.zeros(o.shape, jnp.float32)
    for x, w in zip(xs, ws):
        acc = acc + jax.lax.dot_general(
            x[...], w[...], (((1,), (0,)), ((), ())),
            precision=HI, preferred_element_type=jnp.float32)
    acc = acc + b[...]
    if act == 'relu':
        acc = jnp.maximum(acc, 0.0)
    elif act == 'leaky':
        acc = jnp.where(acc >= 0, acc, 0.01 * acc)
    o[...] = acc


def _mm(xs_ws, b, act=None, bm=2000):
    xs = [x for x, _ in xs_ws]
    ws = [w for _, w in xs_ws]
    m = xs[0].shape[0]
    nout = ws[0].shape[1]
    grid = (m // bm,)
    in_specs = (
        [pl.BlockSpec((bm, x.shape[1]), lambda i: (i, 0)) for x in xs]
        + [pl.BlockSpec(w.shape, lambda i: (0, 0)) for w in ws]
        + [pl.BlockSpec((1, nout), lambda i: (0, 0))]
    )
    return pl.pallas_call(
        functools.partial(_mm_body, len(xs), act),
        grid=grid,
        in_specs=in_specs,
        out_specs=pl.BlockSpec((bm, nout), lambda i: (i, 0)),
        out_shape=jax.ShapeDtypeStruct((m, nout), jnp.float32),
    )(*xs, *ws, b.reshape(1, -1))


def kernel(nodes, edges, senders, receivers, globals_,
           W_gcn1, b_gcn1, W_gcn2, b_gcn2,
           Wq1, bq1, Wl1, bl1,
           Wq2, bq2, Wl2, bl2, Wn2, bn2,
           We, be, Wn, bn, Wg, bg):
    s_idx = senders.astype(jnp.int32)
    r_idx = receivers.astype(jnp.int32)
    sdeg, rdeg = _sc_degrees_call()(s_idx, r_idx)
    ssc = jax.lax.rsqrt(sdeg[:N, None] + 1.0)
    rsc = jax.lax.rsqrt(rdeg[:N, None] + 1.0)

    h = _mm([(nodes, W_gcn1)], b_gcn1, act='relu') * ssc
    h = (h + _sc_spmm(h, s_idx, r_idx)) * rsc
    h = _mm([(h, W_gcn2)], b_gcn2, act='relu') * ssc
    h = (h + _sc_spmm(h, s_idx, r_idx)) * rsc

    q = _mm([(h, Wq1)], bq1)
    wl_s, wl_r, wl_e = Wl1[:256, 0], Wl1[256:512, 0], Wl1[512:, 0]
    a_s = q @ wl_s + bl1[0]
    a_r = q @ wl_r
    w1 = _sc_gatw(a_s, a_r, _edge_base(edges, wl_e), s_idx, r_idx)
    h = _sc_spmm(q, s_idx, r_idx, w1)
    h = jnp.where(h >= 0, h, 0.01 * h)

    q = _mm([(h, Wq2)], bq2)
    wl_s, wl_r, wl_e = Wl2[:256, 0], Wl2[256:512, 0], Wl2[512:, 0]
    a_s = q @ wl_s + bl2[0]
    a_r = q @ wl_r
    w2 = _sc_gatw(a_s, a_r, _edge_base(edges, wl_e), s_idx, r_idx)
    agg = _sc_spmm(q, s_idx, r_idx, w2)
    h = _mm([(agg, Wn2)], bn2)

    We_e, We_s, We_r = We[:16], We[16:272], We[272:528]
    z256 = jnp.zeros((256,), jnp.float32)
    ps2 = _mm2(h, We_s, z256, bm=2000)
    pr2 = _mm2(h, We_r, z256, bm=2000)
    eq2 = _mm2(edges, We_e, be, bm=4000)
    sent_agg, recv_agg = _sc_gnet(ps2, pr2, eq2, s_idx, r_idx)

    Wn_h, Wn_s, Wn_r = Wn[:256], Wn[256:512], Wn[512:768]
    new_nodes = _mm([(h, Wn_h), (sent_agg, Wn_s), (recv_agg, Wn_r)], bn,
                    act='relu')

    node_attr = jnp.sum(new_nodes, axis=0, keepdims=True)
    edge_attr = jnp.sum(sent_agg, axis=0, keepdims=True)
    g = jnp.zeros((1, 64), jnp.float32)
    feats = jnp.concatenate([node_attr, edge_attr, g], axis=1)
    logits = jnp.maximum(feats @ Wg + bg, 0.0)
    return logits
